# baseline (speedup 1.0000x reference)
import functools

import jax
import jax.numpy as jnp
from jax import lax
from jax.experimental import pallas as pl
from jax.experimental.pallas import tpu as pltpu
from jax.experimental.pallas import tpu_sc as plsc

N = 50000
E = 800000
H = 64
MAXZ = 100
L = 4

NC = 2
NS = 16
NW = NC * NS
C = 128

E_PAD = 819200
N_PAD = 50176

_MESH = plsc.VectorSubcoreMesh(
    core_axis_name="c", subcore_axis_name="s", num_cores=NC, num_subcores=NS
)
_SC_PARAMS = pltpu.CompilerParams(use_tc_tiling_on_sc=False)


def _make_gather2(D, dtype):
    rows_per_tile = E_PAD // NW // C

    @functools.partial(
        pl.kernel,
        mesh=_MESH,
        compiler_params=_SC_PARAMS,
        out_type=(
            jax.ShapeDtypeStruct((E_PAD, D), dtype),
            jax.ShapeDtypeStruct((E_PAD, D), dtype),
        ),
        scratch_types=[
            pltpu.VMEM((rows_per_tile, C), jnp.int32),
            pltpu.VMEM((rows_per_tile, C), jnp.int32),
            pltpu.VMEM((C, D), dtype),
            pltpu.VMEM((C, D), dtype),
            pltpu.SemaphoreType.DMA,
            pltpu.SemaphoreType.DMA,
        ],
    )
    def gather2(table, rowm, colm, out_r, out_c, idxr, idxc, bufr, bufc,
                semr, semc):
        wid = lax.axis_index("s") * NC + lax.axis_index("c")
        r0 = wid * rows_per_tile
        pltpu.sync_copy(rowm.at[pl.ds(r0, rows_per_tile)], idxr)
        pltpu.sync_copy(colm.at[pl.ds(r0, rows_per_tile)], idxc)

        def chunk(g, carry):
            e = (r0 + g) * C
            cr = pltpu.async_copy(table.at[idxr.at[g]], bufr, semr)
            cc = pltpu.async_copy(table.at[idxc.at[g]], bufc, semc)
            cr.wait()
            cc.wait()
            pltpu.sync_copy(bufr, out_r.at[pl.ds(e, C)])
            pltpu.sync_copy(bufc, out_c.at[pl.ds(e, C)])
            return carry

        lax.fori_loop(0, rows_per_tile, chunk, 0)

    return gather2


_gather2_h = _make_gather2(H, jnp.bfloat16)
_gather2_pos = _make_gather2(16, jnp.float32)


HW = H // 2
_ROWS_PER_TILE_SC = E_PAD // NS // C
_IDX_WINDOW = 16
_NODES_PER_TILE = N_PAD // NS


@functools.partial(
    pl.kernel,
    mesh=_MESH,
    compiler_params=_SC_PARAMS,
    out_type=(
        jax.ShapeDtypeStruct((N_PAD, HW), jnp.float32),
        jax.ShapeDtypeStruct((N_PAD, HW), jnp.float32),
    ),
    scratch_types=[
        pltpu.VMEM((_IDX_WINDOW, C), jnp.int32),
        pltpu.VMEM((C, HW), jnp.float32),
        pltpu.VMEM_SHARED((N_PAD, HW), jnp.float32),
    ],
)
def _scatter_add(rowm, m0, m1, zeros, agg0, agg1, idxv, mbuf, shared):
    cid = lax.axis_index("c")
    sid = lax.axis_index("s")
    z0 = sid * _NODES_PER_TILE
    pltpu.sync_copy(zeros.at[pl.ds(z0, _NODES_PER_TILE)],
                    shared.at[pl.ds(z0, _NODES_PER_TILE)])
    plsc.subcore_barrier()

    r0 = sid * _ROWS_PER_TILE_SC

    def main(m_hbm):
        def window(w, carry):
            rw = r0 + w * _IDX_WINDOW
            pltpu.sync_copy(rowm.at[pl.ds(rw, _IDX_WINDOW)], idxv)

            def chunk(j, carry2):
                e = (rw + j) * C
                pltpu.sync_copy(m_hbm.at[pl.ds(e, C)], mbuf)
                pltpu.sync_copy(mbuf, shared.at[idxv.at[j]], add=True)
                return carry2

            return lax.fori_loop(0, _IDX_WINDOW, chunk, carry)

        lax.fori_loop(0, _ROWS_PER_TILE_SC // _IDX_WINDOW, window, 0)

    @pl.when(cid == 0)
    def _():
        main(m0)

    @pl.when(cid == 1)
    def _():
        main(m1)

    plsc.subcore_barrier()

    @pl.when(cid == 0)
    def _():
        pltpu.sync_copy(shared.at[pl.ds(z0, _NODES_PER_TILE)],
                        agg0.at[pl.ds(z0, _NODES_PER_TILE)])

    @pl.when(cid == 1)
    def _():
        pltpu.sync_copy(shared.at[pl.ds(z0, _NODES_PER_TILE)],
                        agg1.at[pl.ds(z0, _NODES_PER_TILE)])


BE = 2048
BN = 1024


def _emb_body(z_ref, emb_ref, out_ref, outb_ref):
    z = z_ref[...]
    oh = (z == lax.broadcasted_iota(jnp.int32, (BN, 128), 1)).astype(jnp.float32)
    h = jnp.dot(oh, emb_ref[...], preferred_element_type=jnp.float32)
    out_ref[...] = h
    outb_ref[...] = h.astype(jnp.bfloat16)


def _emb_lookup(z2d, emb_pad):
    return pl.pallas_call(
        _emb_body,
        grid=(N_PAD // BN,),
        in_specs=[
            pl.BlockSpec((BN, 1), lambda i: (i, 0)),
            pl.BlockSpec((128, H), lambda i: (0, 0)),
        ],
        out_specs=(
            pl.BlockSpec((BN, H), lambda i: (i, 0)),
            pl.BlockSpec((BN, H), lambda i: (i, 0)),
        ),
        out_shape=(
            jax.ShapeDtypeStruct((N_PAD, H), jnp.float32),
            jax.ShapeDtypeStruct((N_PAD, H), jnp.bfloat16),
        ),
    )(z2d, emb_pad)


def _dsq_body(pr_ref, pc_ref, out_ref):
    rel = pr_ref[...] - pc_ref[...]
    out_ref[...] = jnp.sum(rel * rel, axis=1, keepdims=True)


def _dsq(pr, pc):
    return pl.pallas_call(
        _dsq_body,
        grid=(E_PAD // BE,),
        in_specs=[
            pl.BlockSpec((BE, 16), lambda i: (i, 0)),
            pl.BlockSpec((BE, 16), lambda i: (i, 0)),
        ],
        out_specs=pl.BlockSpec((BE, 1), lambda i: (i, 0)),
        out_shape=jax.ShapeDtypeStruct((E_PAD, 1), jnp.float32),
    )(pr, pc)


def _silu(x):
    return x * jax.nn.sigmoid(x)


def _edge_body(hr_ref, hc_ref, dsq_ref, w1r_ref, w1c_ref, w1d_ref, b1_ref,
               w2_ref, b2_ref, m0_ref, m1_ref):
    p = pl.program_id(0)
    t = (jnp.dot(hr_ref[...], w1r_ref[...], preferred_element_type=jnp.float32)
         + jnp.dot(hc_ref[...], w1c_ref[...], preferred_element_type=jnp.float32)
         + dsq_ref[...] * w1d_ref[...]
         + b1_ref[...])
    t = _silu(t)
    t = jnp.dot(t, w2_ref[...], preferred_element_type=jnp.float32) + b2_ref[...]
    m = _silu(t)
    rows = p * BE + lax.broadcasted_iota(jnp.int32, (BE, 1), 0)
    m = jnp.where(rows < E, m, 0.0)
    m0_ref[...] = m[:, :HW]
    m1_ref[...] = m[:, HW:]


def _edge_mlp(hr, hc, dsq, w1r, w1c, w1d, b1, w2, b2):
    wspec = lambda shape: pl.BlockSpec(shape, lambda i: (0, 0))
    return pl.pallas_call(
        _edge_body,
        grid=(E_PAD // BE,),
        in_specs=[
            pl.BlockSpec((BE, H), lambda i: (i, 0)),
            pl.BlockSpec((BE, H), lambda i: (i, 0)),
            pl.BlockSpec((BE, 1), lambda i: (i, 0)),
            wspec((H, H)), wspec((H, H)), wspec((1, H)), wspec((1, H)),
            wspec((H, H)), wspec((1, H)),
        ],
        out_specs=(
            pl.BlockSpec((BE, HW), lambda i: (i, 0)),
            pl.BlockSpec((BE, HW), lambda i: (i, 0)),
        ),
        out_shape=(
            jax.ShapeDtypeStruct((E_PAD, HW), jnp.float32),
            jax.ShapeDtypeStruct((E_PAD, HW), jnp.float32),
        ),
    )(hr, hc, dsq, w1r, w1c, w1d, b1, w2, b2)


def _node_body(h_ref, a0_ref, a1_ref, wh_ref, wa0_ref, wa1_ref, b1_ref,
               w2_ref, b2_ref, g_ref, bb_ref, out_ref, outb_ref):
    h = h_ref[...]
    t = (jnp.dot(h, wh_ref[...], preferred_element_type=jnp.float32)
         + jnp.dot(a0_ref[...], wa0_ref[...], preferred_element_type=jnp.float32)
         + jnp.dot(a1_ref[...], wa1_ref[...], preferred_element_type=jnp.float32)
         + b1_ref[...])
    t = _silu(t)
    hn = jnp.dot(t, w2_ref[...], preferred_element_type=jnp.float32) + b2_ref[...]
    y = h + hn
    mu = jnp.mean(y, axis=-1, keepdims=True)
    d = y - mu
    var = jnp.mean(d * d, axis=-1, keepdims=True)
    hh = d * lax.rsqrt(var + 1e-5) * g_ref[...] + bb_ref[...]
    out_ref[...] = hh
    outb_ref[...] = hh.astype(jnp.bfloat16)


def _node_mlp(h, a0, a1, wh, wa0, wa1, b1, w2, b2, g, bb):
    wspec = lambda shape: pl.BlockSpec(shape, lambda i: (0, 0))
    return pl.pallas_call(
        _node_body,
        grid=(N_PAD // BN,),
        in_specs=[
            pl.BlockSpec((BN, H), lambda i: (i, 0)),
            pl.BlockSpec((BN, HW), lambda i: (i, 0)),
            pl.BlockSpec((BN, HW), lambda i: (i, 0)),
            wspec((H, H)), wspec((HW, H)), wspec((HW, H)), wspec((1, H)),
            wspec((H, H)), wspec((1, H)), wspec((1, H)), wspec((1, H)),
        ],
        out_specs=(
            pl.BlockSpec((BN, H), lambda i: (i, 0)),
            pl.BlockSpec((BN, H), lambda i: (i, 0)),
        ),
        out_shape=(
            jax.ShapeDtypeStruct((N_PAD, H), jnp.float32),
            jax.ShapeDtypeStruct((N_PAD, H), jnp.bfloat16),
        ),
    )(h, a0, a1, wh, wa0, wa1, b1, w2, b2, g, bb)


def kernel(z, pos, batch, edge_index, params):
    f32 = jnp.float32
    row = edge_index[0].astype(jnp.int32)
    col = edge_index[1].astype(jnp.int32)
    pad_e = E_PAD - E
    pad_idx = jnp.arange(pad_e, dtype=jnp.int32) % N
    row_p = jnp.concatenate([row, pad_idx]).reshape(E_PAD // C, C)
    col_p = jnp.concatenate([col, pad_idx]).reshape(E_PAD // C, C)

    z_p = jnp.concatenate(
        [z.astype(jnp.int32), jnp.zeros((N_PAD - N,), jnp.int32)]
    ).reshape(N_PAD, 1)
    pos16 = jnp.zeros((N_PAD, 16), f32).at[:N, :3].set(pos.astype(f32))
    emb_pad = jnp.zeros((128, H), f32).at[:MAXZ].set(params["emb"].astype(f32))
    zeros_half = jnp.zeros((N_PAD, HW), f32)

    h, hb = _emb_lookup(z_p, emb_pad)
    pr, pc = _gather2_pos(pos16, row_p, col_p)
    dsq = _dsq(pr, pc)

    for i in range(L):
        p = params[f"l{i}"]
        w1r = p["eW1"][:H].astype(jnp.bfloat16)
        w1c = p["eW1"][H:2 * H].astype(jnp.bfloat16)
        w1d = p["eW1"][2 * H:].reshape(1, H)
        b1 = p["eb1"].reshape(1, H)
        b2 = p["eb2"].reshape(1, H)

        hr, hc = _gather2_h(hb, row_p, col_p)
        m0, m1 = _edge_mlp(hr, hc, dsq, w1r, w1c, w1d, b1,
                           p["eW2"].astype(jnp.bfloat16), b2)
        a0, a1 = _scatter_add(row_p, m0, m1, zeros_half)

        wh = p["nW1"][:H]
        wa0 = p["nW1"][H:H + HW]
        wa1 = p["nW1"][H + HW:]
        h, hb = _node_mlp(h, a0, a1, wh, wa0, wa1, p["nb1"].reshape(1, H),
                      p["nW2"], p["nb2"].reshape(1, H), p["g"].reshape(1, H),
                      p["b"].reshape(1, H))

    return h[:N]


# speedup vs baseline: 1.1967x; 1.1967x over previous
import functools

import jax
import jax.numpy as jnp
from jax import lax
from jax.experimental import pallas as pl
from jax.experimental.pallas import tpu as pltpu
from jax.experimental.pallas import tpu_sc as plsc

N = 50000
E = 800000
H = 64
MAXZ = 100
L = 4

NC = 2
NS = 16
NW = NC * NS
C = 128

E_PAD = 819200
N_PAD = 50176

_MESH = plsc.VectorSubcoreMesh(
    core_axis_name="c", subcore_axis_name="s", num_cores=NC, num_subcores=NS
)
_SC_PARAMS = pltpu.CompilerParams(use_tc_tiling_on_sc=False)


def _make_gather2(D):
    R = E_PAD // NW // C

    @functools.partial(
        pl.kernel,
        mesh=_MESH,
        compiler_params=_SC_PARAMS,
        out_type=(
            jax.ShapeDtypeStruct((E_PAD, D), jnp.float32),
            jax.ShapeDtypeStruct((E_PAD, D), jnp.float32),
        ),
        scratch_types=[
            pltpu.VMEM((R, C), jnp.int32),
            pltpu.VMEM((R, C), jnp.int32),
            pltpu.VMEM((C, D), jnp.float32),
            pltpu.VMEM((C, D), jnp.float32),
            pltpu.VMEM((C, D), jnp.float32),
            pltpu.VMEM((C, D), jnp.float32),
            pltpu.SemaphoreType.DMA,
            pltpu.SemaphoreType.DMA,
            pltpu.SemaphoreType.DMA,
            pltpu.SemaphoreType.DMA,
        ],
    )
    def gather2(table, rowm, colm, out_r, out_c, idxr, idxc,
                br0, bc0, br1, bc1, sr0, sc0, sr1, sc1):
        wid = lax.axis_index("s") * NC + lax.axis_index("c")
        r0 = wid * R
        pltpu.sync_copy(rowm.at[pl.ds(r0, R)], idxr)
        pltpu.sync_copy(colm.at[pl.ds(r0, R)], idxc)

        def issue(g, br, bc, sr, sc):
            pltpu.async_copy(table.at[idxr.at[g]], br, sr)
            pltpu.async_copy(table.at[idxc.at[g]], bc, sc)

        def drain(br, bc, sr, sc):
            pltpu.make_async_copy(table.at[idxr.at[0]], br, sr).wait()
            pltpu.make_async_copy(table.at[idxc.at[0]], bc, sc).wait()

        issue(0, br0, bc0, sr0, sc0)

        def body(g2, carry):
            g = g2 * 2
            issue(g + 1, br1, bc1, sr1, sc1)
            drain(br0, bc0, sr0, sc0)
            e = (r0 + g) * C
            pltpu.sync_copy(br0, out_r.at[pl.ds(e, C)])
            pltpu.sync_copy(bc0, out_c.at[pl.ds(e, C)])

            @pl.when(g + 2 < R)
            def _():
                issue(g + 2, br0, bc0, sr0, sc0)

            drain(br1, bc1, sr1, sc1)
            e1 = (r0 + g + 1) * C
            pltpu.sync_copy(br1, out_r.at[pl.ds(e1, C)])
            pltpu.sync_copy(bc1, out_c.at[pl.ds(e1, C)])
            return carry

        lax.fori_loop(0, R // 2, body, 0)

    return gather2


_gather2_h = _make_gather2(H)
_gather2_pos = _make_gather2(16)


HW = H // 2
_RSC = E_PAD // NS // C
_IDXW = 40
_NODES_PER_TILE = N_PAD // NS


@functools.partial(
    pl.kernel,
    mesh=_MESH,
    compiler_params=_SC_PARAMS,
    out_type=(
        jax.ShapeDtypeStruct((N_PAD, HW), jnp.float32),
        jax.ShapeDtypeStruct((N_PAD, HW), jnp.float32),
    ),
    scratch_types=[
        pltpu.VMEM((_IDXW, C), jnp.int32),
        pltpu.VMEM((C, HW), jnp.float32),
        pltpu.VMEM((C, HW), jnp.float32),
        pltpu.VMEM_SHARED((N_PAD, HW), jnp.float32),
        pltpu.SemaphoreType.DMA,
        pltpu.SemaphoreType.DMA,
    ],
)
def _scatter_add(rowm, m0, m1, zeros, agg0, agg1, idxv, mb0, mb1, shared,
                 sm0, sm1):
    cid = lax.axis_index("c")
    sid = lax.axis_index("s")
    z0 = sid * _NODES_PER_TILE
    pltpu.sync_copy(zeros.at[pl.ds(z0, _NODES_PER_TILE)],
                    shared.at[pl.ds(z0, _NODES_PER_TILE)])
    plsc.subcore_barrier()

    r0 = sid * _RSC

    def main(m_hbm):
        def window(w, carry):
            rw = r0 + w * _IDXW
            pltpu.sync_copy(rowm.at[pl.ds(rw, _IDXW)], idxv)
            pltpu.async_copy(m_hbm.at[pl.ds(rw * C, C)], mb0, sm0)

            def chunk2(j2, carry2):
                j = j2 * 2
                pltpu.async_copy(m_hbm.at[pl.ds((rw + j + 1) * C, C)],
                                 mb1, sm1)
                pltpu.make_async_copy(m_hbm.at[pl.ds(0, C)], mb0, sm0).wait()
                pltpu.sync_copy(mb0, shared.at[idxv.at[j]], add=True)

                @pl.when(j + 2 < _IDXW)
                def _():
                    pltpu.async_copy(m_hbm.at[pl.ds((rw + j + 2) * C, C)],
                                     mb0, sm0)

                pltpu.make_async_copy(m_hbm.at[pl.ds(0, C)], mb1, sm1).wait()
                pltpu.sync_copy(mb1, shared.at[idxv.at[j + 1]], add=True)
                return carry2

            return lax.fori_loop(0, _IDXW // 2, chunk2, carry)

        lax.fori_loop(0, _RSC // _IDXW, window, 0)

    @pl.when(cid == 0)
    def _():
        main(m0)

    @pl.when(cid == 1)
    def _():
        main(m1)

    plsc.subcore_barrier()

    @pl.when(cid == 0)
    def _():
        pltpu.sync_copy(shared.at[pl.ds(z0, _NODES_PER_TILE)],
                        agg0.at[pl.ds(z0, _NODES_PER_TILE)])

    @pl.when(cid == 1)
    def _():
        pltpu.sync_copy(shared.at[pl.ds(z0, _NODES_PER_TILE)],
                        agg1.at[pl.ds(z0, _NODES_PER_TILE)])


BE = 2048
BN = 1024


def _emb_body(z_ref, emb_ref, out_ref):
    z = z_ref[...]
    oh = (z == lax.broadcasted_iota(jnp.int32, (BN, 128), 1)).astype(jnp.float32)
    out_ref[...] = jnp.dot(oh, emb_ref[...], preferred_element_type=jnp.float32)


def _emb_lookup(z2d, emb_pad):
    return pl.pallas_call(
        _emb_body,
        grid=(N_PAD // BN,),
        in_specs=[
            pl.BlockSpec((BN, 1), lambda i: (i, 0)),
            pl.BlockSpec((128, H), lambda i: (0, 0)),
        ],
        out_specs=pl.BlockSpec((BN, H), lambda i: (i, 0)),
        out_shape=jax.ShapeDtypeStruct((N_PAD, H), jnp.float32),
    )(z2d, emb_pad)


def _dsq_body(pr_ref, pc_ref, out_ref):
    rel = pr_ref[...] - pc_ref[...]
    out_ref[...] = jnp.sum(rel * rel, axis=1, keepdims=True)


def _dsq(pr, pc):
    return pl.pallas_call(
        _dsq_body,
        grid=(E_PAD // BE,),
        in_specs=[
            pl.BlockSpec((BE, 16), lambda i: (i, 0)),
            pl.BlockSpec((BE, 16), lambda i: (i, 0)),
        ],
        out_specs=pl.BlockSpec((BE, 1), lambda i: (i, 0)),
        out_shape=jax.ShapeDtypeStruct((E_PAD, 1), jnp.float32),
    )(pr, pc)


def _silu(x):
    return x * jax.nn.sigmoid(x)


def _bdot(a, b_ref):
    return jnp.dot(a.astype(jnp.bfloat16), b_ref[...],
                   preferred_element_type=jnp.float32)


def _edge_body(hr_ref, hc_ref, dsq_ref, w1r_ref, w1c_ref, w1d_ref, b1_ref,
               w2_ref, b2_ref, m0_ref, m1_ref):
    p = pl.program_id(0)
    t = (_bdot(hr_ref[...], w1r_ref) + _bdot(hc_ref[...], w1c_ref)
         + dsq_ref[...] * w1d_ref[...]
         + b1_ref[...])
    t = _silu(t)
    t = _bdot(t, w2_ref) + b2_ref[...]
    m = _silu(t)
    rows = p * BE + lax.broadcasted_iota(jnp.int32, (BE, 1), 0)
    m = jnp.where(rows < E, m, 0.0)
    m0_ref[...] = m[:, :HW]
    m1_ref[...] = m[:, HW:]


def _edge_mlp(hr, hc, dsq, w1r, w1c, w1d, b1, w2, b2):
    wspec = lambda shape: pl.BlockSpec(shape, lambda i: (0, 0))
    return pl.pallas_call(
        _edge_body,
        grid=(E_PAD // BE,),
        in_specs=[
            pl.BlockSpec((BE, H), lambda i: (i, 0)),
            pl.BlockSpec((BE, H), lambda i: (i, 0)),
            pl.BlockSpec((BE, 1), lambda i: (i, 0)),
            wspec((H, H)), wspec((H, H)), wspec((1, H)), wspec((1, H)),
            wspec((H, H)), wspec((1, H)),
        ],
        out_specs=(
            pl.BlockSpec((BE, HW), lambda i: (i, 0)),
            pl.BlockSpec((BE, HW), lambda i: (i, 0)),
        ),
        out_shape=(
            jax.ShapeDtypeStruct((E_PAD, HW), jnp.float32),
            jax.ShapeDtypeStruct((E_PAD, HW), jnp.float32),
        ),
    )(hr, hc, dsq, w1r, w1c, w1d, b1, w2, b2)


def _node_body(h_ref, a0_ref, a1_ref, wh_ref, wa0_ref, wa1_ref, b1_ref,
               w2_ref, b2_ref, g_ref, bb_ref, out_ref):
    h = h_ref[...]
    t = (jnp.dot(h, wh_ref[...], preferred_element_type=jnp.float32)
         + jnp.dot(a0_ref[...], wa0_ref[...], preferred_element_type=jnp.float32)
         + jnp.dot(a1_ref[...], wa1_ref[...], preferred_element_type=jnp.float32)
         + b1_ref[...])
    t = _silu(t)
    hn = jnp.dot(t, w2_ref[...], preferred_element_type=jnp.float32) + b2_ref[...]
    y = h + hn
    mu = jnp.mean(y, axis=-1, keepdims=True)
    d = y - mu
    var = jnp.mean(d * d, axis=-1, keepdims=True)
    out_ref[...] = d * lax.rsqrt(var + 1e-5) * g_ref[...] + bb_ref[...]


def _node_mlp(h, a0, a1, wh, wa0, wa1, b1, w2, b2, g, bb):
    wspec = lambda shape: pl.BlockSpec(shape, lambda i: (0, 0))
    return pl.pallas_call(
        _node_body,
        grid=(N_PAD // BN,),
        in_specs=[
            pl.BlockSpec((BN, H), lambda i: (i, 0)),
            pl.BlockSpec((BN, HW), lambda i: (i, 0)),
            pl.BlockSpec((BN, HW), lambda i: (i, 0)),
            wspec((H, H)), wspec((HW, H)), wspec((HW, H)), wspec((1, H)),
            wspec((H, H)), wspec((1, H)), wspec((1, H)), wspec((1, H)),
        ],
        out_specs=pl.BlockSpec((BN, H), lambda i: (i, 0)),
        out_shape=jax.ShapeDtypeStruct((N_PAD, H), jnp.float32),
    )(h, a0, a1, wh, wa0, wa1, b1, w2, b2, g, bb)


def kernel(z, pos, batch, edge_index, params):
    f32 = jnp.float32
    bf16 = jnp.bfloat16
    row = edge_index[0].astype(jnp.int32)
    col = edge_index[1].astype(jnp.int32)
    pad_e = E_PAD - E
    pad_idx = jnp.arange(pad_e, dtype=jnp.int32) % N
    row_p = jnp.concatenate([row, pad_idx]).reshape(E_PAD // C, C)
    col_p = jnp.concatenate([col, pad_idx]).reshape(E_PAD // C, C)

    z_p = jnp.concatenate(
        [z.astype(jnp.int32), jnp.zeros((N_PAD - N,), jnp.int32)]
    ).reshape(N_PAD, 1)
    pos16 = jnp.zeros((N_PAD, 16), f32).at[:N, :3].set(pos.astype(f32))
    emb_pad = jnp.zeros((128, H), f32).at[:MAXZ].set(params["emb"].astype(f32))
    zeros_half = jnp.zeros((N_PAD, HW), f32)

    h = _emb_lookup(z_p, emb_pad)
    pr, pc = _gather2_pos(pos16, row_p, col_p)
    dsq = _dsq(pr, pc)

    for i in range(L):
        p = params[f"l{i}"]
        w1r = p["eW1"][:H].astype(bf16)
        w1c = p["eW1"][H:2 * H].astype(bf16)
        w1d = p["eW1"][2 * H:].reshape(1, H)
        b1 = p["eb1"].reshape(1, H)
        b2 = p["eb2"].reshape(1, H)

        hr, hc = _gather2_h(h, row_p, col_p)
        m0, m1 = _edge_mlp(hr, hc, dsq, w1r, w1c, w1d, b1,
                           p["eW2"].astype(bf16), b2)
        a0, a1 = _scatter_add(row_p, m0, m1, zeros_half)

        wh = p["nW1"][:H]
        wa0 = p["nW1"][H:H + HW]
        wa1 = p["nW1"][H + HW:]
        h = _node_mlp(h, a0, a1, wh, wa0, wa1, p["nb1"].reshape(1, H),
                      p["nW2"], p["nb2"].reshape(1, H), p["g"].reshape(1, H),
                      p["b"].reshape(1, H))

    return h[:N]


# speedup vs baseline: 2.2847x; 1.9092x over previous
import functools

import jax
import jax.numpy as jnp
from jax import lax
from jax.experimental import pallas as pl
from jax.experimental.pallas import tpu as pltpu
from jax.experimental.pallas import tpu_sc as plsc

N = 50000
E = 800000
H = 64
MAXZ = 100
L = 4

NC = 2
NS = 16
NW = NC * NS
C = 128

E_PAD = 819200
N_PAD = 50176

_MESH = plsc.VectorSubcoreMesh(
    core_axis_name="c", subcore_axis_name="s", num_cores=NC, num_subcores=NS
)
_SC_PARAMS = pltpu.CompilerParams(use_tc_tiling_on_sc=False)


def _make_gather2(D):
    R = E_PAD // NW // C
    out_shape = (E_PAD, D)

    @functools.partial(
        pl.kernel,
        mesh=_MESH,
        compiler_params=_SC_PARAMS,
        out_type=(
            jax.ShapeDtypeStruct(out_shape, jnp.float32),
            jax.ShapeDtypeStruct(out_shape, jnp.float32),
        ),
        scratch_types=[
            pltpu.VMEM((R, C), jnp.int32),
            pltpu.VMEM((R, C), jnp.int32),
            pltpu.VMEM((C, D), jnp.float32),
            pltpu.VMEM((C, D), jnp.float32),
            pltpu.VMEM((C, D), jnp.float32),
            pltpu.VMEM((C, D), jnp.float32),
            pltpu.SemaphoreType.DMA,
            pltpu.SemaphoreType.DMA,
            pltpu.SemaphoreType.DMA,
            pltpu.SemaphoreType.DMA,
        ],
    )
    def gather2(table, rowm, colm, out_r, out_c, idxr, idxc,
                br0, bc0, br1, bc1, sr0, sc0, sr1, sc1):
        wid = lax.axis_index("s") * NC + lax.axis_index("c")
        r0 = wid * R
        pltpu.sync_copy(rowm.at[pl.ds(r0, R)], idxr)
        pltpu.sync_copy(colm.at[pl.ds(r0, R)], idxc)

        def issue(g, br, bc, sr, sc):
            pltpu.async_copy(table.at[idxr.at[g]], br, sr)
            pltpu.async_copy(table.at[idxc.at[g]], bc, sc)

        def drain(br, bc, sr, sc):
            pltpu.make_async_copy(table.at[idxr.at[0]], br, sr).wait()
            pltpu.make_async_copy(table.at[idxc.at[0]], bc, sc).wait()

        issue(0, br0, bc0, sr0, sc0)

        def body(g2, carry):
            g = g2 * 2
            issue(g + 1, br1, bc1, sr1, sc1)
            drain(br0, bc0, sr0, sc0)
            e = (r0 + g) * C
            pltpu.sync_copy(br0, out_r.at[pl.ds(e, C)])
            pltpu.sync_copy(bc0, out_c.at[pl.ds(e, C)])

            @pl.when(g + 2 < R)
            def _():
                issue(g + 2, br0, bc0, sr0, sc0)

            drain(br1, bc1, sr1, sc1)
            e1 = (r0 + g + 1) * C
            pltpu.sync_copy(br1, out_r.at[pl.ds(e1, C)])
            pltpu.sync_copy(bc1, out_c.at[pl.ds(e1, C)])
            return carry

        lax.fori_loop(0, R // 2, body, 0)

    return gather2


_gather2_h = _make_gather2(H)
_gather2_pos = _make_gather2(16)


HW = H // 2
_RSC = E_PAD // NS // C
_IDXW = 40
_NODES_PER_TILE = N_PAD // NS


@functools.partial(
    pl.kernel,
    mesh=_MESH,
    compiler_params=_SC_PARAMS,
    out_type=(
        jax.ShapeDtypeStruct((N_PAD, HW), jnp.float32),
        jax.ShapeDtypeStruct((N_PAD, HW), jnp.float32),
    ),
    scratch_types=[
        pltpu.VMEM((_IDXW, C), jnp.int32),
        pltpu.VMEM((C, HW), jnp.float32),
        pltpu.VMEM((C, HW), jnp.float32),
        pltpu.VMEM_SHARED((N_PAD, HW), jnp.float32),
        pltpu.SemaphoreType.DMA,
        pltpu.SemaphoreType.DMA,
    ],
)
def _scatter_add(mv, rowm, zeros, agg0, agg1, idxv, mb0, mb1, shared,
                 sm0, sm1):
    cid = lax.axis_index("c")
    sid = lax.axis_index("s")
    z0 = sid * _NODES_PER_TILE
    pltpu.sync_copy(zeros.at[pl.ds(z0, _NODES_PER_TILE)],
                    shared.at[pl.ds(z0, _NODES_PER_TILE)])
    plsc.subcore_barrier()

    r0 = sid * _RSC

    def main(coff):
        def window(w, carry):
            rw = r0 + w * _IDXW
            pltpu.sync_copy(rowm.at[pl.ds(rw, _IDXW)], idxv)
            pltpu.async_copy(mv.at[pl.ds(rw * C, C), pl.ds(coff, HW)],
                             mb0, sm0)

            def chunk2(j2, carry2):
                j = j2 * 2
                pltpu.async_copy(
                    mv.at[pl.ds((rw + j + 1) * C, C), pl.ds(coff, HW)],
                    mb1, sm1)
                pltpu.make_async_copy(
                    mv.at[pl.ds(0, C), pl.ds(coff, HW)], mb0, sm0).wait()
                pltpu.sync_copy(mb0, shared.at[idxv.at[j]], add=True)

                @pl.when(j + 2 < _IDXW)
                def _():
                    pltpu.async_copy(
                        mv.at[pl.ds((rw + j + 2) * C, C), pl.ds(coff, HW)],
                        mb0, sm0)

                pltpu.make_async_copy(
                    mv.at[pl.ds(0, C), pl.ds(coff, HW)], mb1, sm1).wait()
                pltpu.sync_copy(mb1, shared.at[idxv.at[j + 1]], add=True)
                return carry2

            return lax.fori_loop(0, _IDXW // 2, chunk2, carry)

        lax.fori_loop(0, _RSC // _IDXW, window, 0)

    @pl.when(cid == 0)
    def _():
        main(0)

    @pl.when(cid == 1)
    def _():
        main(HW)

    plsc.subcore_barrier()

    @pl.when(cid == 0)
    def _():
        pltpu.sync_copy(shared.at[pl.ds(z0, _NODES_PER_TILE)],
                        agg0.at[pl.ds(z0, _NODES_PER_TILE)])

    @pl.when(cid == 1)
    def _():
        pltpu.sync_copy(shared.at[pl.ds(z0, _NODES_PER_TILE)],
                        agg1.at[pl.ds(z0, _NODES_PER_TILE)])


BE = 2048
BN = 1024


def _emb_body(z_ref, emb_ref, out_ref):
    z = z_ref[...]
    oh = (z == lax.broadcasted_iota(jnp.int32, (BN, 128), 1)).astype(jnp.float32)
    out_ref[...] = jnp.dot(oh, emb_ref[...], preferred_element_type=jnp.float32)


def _emb_lookup(z2d, emb_pad):
    return pl.pallas_call(
        _emb_body,
        grid=(N_PAD // BN,),
        in_specs=[
            pl.BlockSpec((BN, 1), lambda i: (i, 0)),
            pl.BlockSpec((128, H), lambda i: (0, 0)),
        ],
        out_specs=pl.BlockSpec((BN, H), lambda i: (i, 0)),
        out_shape=jax.ShapeDtypeStruct((N_PAD, H), jnp.float32),
    )(z2d, emb_pad)


def _dsq_body(pr_ref, pc_ref, out_ref):
    rel = pr_ref[...] - pc_ref[...]
    out_ref[...] = jnp.sum(rel * rel, axis=1, keepdims=True)


def _dsq(pr, pc):
    return pl.pallas_call(
        _dsq_body,
        grid=(E_PAD // BE,),
        in_specs=[
            pl.BlockSpec((BE, 16), lambda i: (i, 0)),
            pl.BlockSpec((BE, 16), lambda i: (i, 0)),
        ],
        out_specs=pl.BlockSpec((BE, 1), lambda i: (i, 0)),
        out_shape=jax.ShapeDtypeStruct((E_PAD, 1), jnp.float32),
    )(pr, pc)


def _silu(x):
    return x * jax.nn.sigmoid(x)


def _bdot(a, b_ref):
    return jnp.dot(a.astype(jnp.bfloat16), b_ref[...],
                   preferred_element_type=jnp.float32)


BE2 = BE // 2


def _edge_body(hrp_ref, hcp_ref, dsqp_ref, w1rp_ref, w1cp_ref, w1d0_ref,
               w1d1_ref, b1p_ref, w2p_ref, b2p_ref, mp_ref):
    p = pl.program_id(0)
    dsqp = dsqp_ref[...]
    t = (_bdot(hrp_ref[...], w1rp_ref) + _bdot(hcp_ref[...], w1cp_ref)
         + dsqp[:, 0:1] * w1d0_ref[...]
         + dsqp[:, 1:2] * w1d1_ref[...]
         + b1p_ref[...])
    t = _silu(t)
    t = _bdot(t, w2p_ref) + b2p_ref[...]
    m = _silu(t)
    eid = (p * BE + 2 * lax.broadcasted_iota(jnp.int32, (BE2, 2 * H), 0)
           + (lax.broadcasted_iota(jnp.int32, (BE2, 2 * H), 1) >= H)
           .astype(jnp.int32))
    mp_ref[...] = jnp.where(eid < E, m, 0.0)


def _edge_mlp(hrp, hcp, dsqp, w1rp, w1cp, w1d0, w1d1, b1p, w2p, b2p):
    wspec = lambda shape: pl.BlockSpec(shape, lambda i: (0, 0))
    return pl.pallas_call(
        _edge_body,
        grid=(E_PAD // BE,),
        in_specs=[
            pl.BlockSpec((BE2, 2 * H), lambda i: (i, 0)),
            pl.BlockSpec((BE2, 2 * H), lambda i: (i, 0)),
            pl.BlockSpec((BE2, 2), lambda i: (i, 0)),
            wspec((2 * H, 2 * H)), wspec((2 * H, 2 * H)),
            wspec((1, 2 * H)), wspec((1, 2 * H)), wspec((1, 2 * H)),
            wspec((2 * H, 2 * H)), wspec((1, 2 * H)),
        ],
        out_specs=pl.BlockSpec((BE2, 2 * H), lambda i: (i, 0)),
        out_shape=jax.ShapeDtypeStruct((E_PAD // 2, 2 * H), jnp.float32),
    )(hrp, hcp, dsqp, w1rp, w1cp, w1d0, w1d1, b1p, w2p, b2p)


def _node_body(h_ref, a0_ref, a1_ref, wh_ref, wa0_ref, wa1_ref, b1_ref,
               w2_ref, b2_ref, g_ref, bb_ref, out_ref):
    h = h_ref[...]
    t = (jnp.dot(h, wh_ref[...], preferred_element_type=jnp.float32)
         + jnp.dot(a0_ref[...], wa0_ref[...], preferred_element_type=jnp.float32)
         + jnp.dot(a1_ref[...], wa1_ref[...], preferred_element_type=jnp.float32)
         + b1_ref[...])
    t = _silu(t)
    hn = jnp.dot(t, w2_ref[...], preferred_element_type=jnp.float32) + b2_ref[...]
    y = h + hn
    mu = jnp.mean(y, axis=-1, keepdims=True)
    d = y - mu
    var = jnp.mean(d * d, axis=-1, keepdims=True)
    out_ref[...] = d * lax.rsqrt(var + 1e-5) * g_ref[...] + bb_ref[...]


def _node_mlp(h, a0, a1, wh, wa0, wa1, b1, w2, b2, g, bb):
    wspec = lambda shape: pl.BlockSpec(shape, lambda i: (0, 0))
    return pl.pallas_call(
        _node_body,
        grid=(N_PAD // BN,),
        in_specs=[
            pl.BlockSpec((BN, H), lambda i: (i, 0)),
            pl.BlockSpec((BN, HW), lambda i: (i, 0)),
            pl.BlockSpec((BN, HW), lambda i: (i, 0)),
            wspec((H, H)), wspec((HW, H)), wspec((HW, H)), wspec((1, H)),
            wspec((H, H)), wspec((1, H)), wspec((1, H)), wspec((1, H)),
        ],
        out_specs=pl.BlockSpec((BN, H), lambda i: (i, 0)),
        out_shape=jax.ShapeDtypeStruct((N_PAD, H), jnp.float32),
    )(h, a0, a1, wh, wa0, wa1, b1, w2, b2, g, bb)


def kernel(z, pos, batch, edge_index, params):
    f32 = jnp.float32
    bf16 = jnp.bfloat16
    row = edge_index[0].astype(jnp.int32)
    col = edge_index[1].astype(jnp.int32)
    pad_e = E_PAD - E
    pad_idx = jnp.arange(pad_e, dtype=jnp.int32) % N
    row_p = jnp.concatenate([row, pad_idx]).reshape(E_PAD // C, C)
    col_p = jnp.concatenate([col, pad_idx]).reshape(E_PAD // C, C)

    z_p = jnp.concatenate(
        [z.astype(jnp.int32), jnp.zeros((N_PAD - N,), jnp.int32)]
    ).reshape(N_PAD, 1)
    pos16 = jnp.zeros((N_PAD, 16), f32).at[:N, :3].set(pos.astype(f32))
    emb_pad = jnp.zeros((128, H), f32).at[:MAXZ].set(params["emb"].astype(f32))
    zeros_half = jnp.zeros((N_PAD, HW), f32)

    h = _emb_lookup(z_p, emb_pad)
    pr, pc = _gather2_pos(pos16, row_p, col_p)
    dsq = _dsq(pr, pc)

    dsqp = dsq.reshape(E_PAD // 2, 2)

    def blockdiag2(w):
        z = jnp.zeros((2 * H, 2 * H), f32)
        return z.at[:H, :H].set(w).at[H:, H:].set(w).astype(bf16)

    for i in range(L):
        p = params[f"l{i}"]
        w1rp = blockdiag2(p["eW1"][:H])
        w1cp = blockdiag2(p["eW1"][H:2 * H])
        w1d = p["eW1"][2 * H:].reshape(1, H)
        zpad = jnp.zeros((1, H), f32)
        w1d0 = jnp.concatenate([w1d, zpad], axis=1)
        w1d1 = jnp.concatenate([zpad, w1d], axis=1)
        b1p = jnp.tile(p["eb1"].reshape(1, H), (1, 2))
        b2p = jnp.tile(p["eb2"].reshape(1, H), (1, 2))

        hr, hc = _gather2_h(h, row_p, col_p)
        hrp = hr.reshape(E_PAD // 2, 2 * H)
        hcp = hc.reshape(E_PAD // 2, 2 * H)
        mp = _edge_mlp(hrp, hcp, dsqp, w1rp, w1cp, w1d0, w1d1, b1p,
                       blockdiag2(p["eW2"]), b2p)
        a0, a1 = _scatter_add(mp.reshape(E_PAD, H), row_p, zeros_half)

        wh = p["nW1"][:H]
        wa0 = p["nW1"][H:H + HW]
        wa1 = p["nW1"][H + HW:]
        h = _node_mlp(h, a0, a1, wh, wa0, wa1, p["nb1"].reshape(1, H),
                      p["nW2"], p["nb2"].reshape(1, H), p["g"].reshape(1, H),
                      p["b"].reshape(1, H))

    return h[:N]


# speedup vs baseline: 2.9458x; 1.2894x over previous
import functools

import jax
import jax.numpy as jnp
from jax import lax
from jax.experimental import pallas as pl
from jax.experimental.pallas import tpu as pltpu
from jax.experimental.pallas import tpu_sc as plsc

N = 50000
E = 800000
H = 64
MAXZ = 100
L = 4

NC = 2
NS = 16
NW = NC * NS
C = 128

E_PAD = 819200
N_PAD = 50176

_MESH = plsc.VectorSubcoreMesh(
    core_axis_name="c", subcore_axis_name="s", num_cores=NC, num_subcores=NS
)
_SC_PARAMS = pltpu.CompilerParams(use_tc_tiling_on_sc=False)


def _make_gather2(D):
    R = E_PAD // NW // C
    out_shape = (E_PAD, D)

    @functools.partial(
        pl.kernel,
        mesh=_MESH,
        compiler_params=_SC_PARAMS,
        out_type=(
            jax.ShapeDtypeStruct(out_shape, jnp.float32),
            jax.ShapeDtypeStruct(out_shape, jnp.float32),
        ),
        scratch_types=[
            pltpu.VMEM((R, C), jnp.int32),
            pltpu.VMEM((R, C), jnp.int32),
            pltpu.VMEM((C, D), jnp.float32),
            pltpu.VMEM((C, D), jnp.float32),
            pltpu.VMEM((C, D), jnp.float32),
            pltpu.VMEM((C, D), jnp.float32),
            pltpu.SemaphoreType.DMA,
            pltpu.SemaphoreType.DMA,
            pltpu.SemaphoreType.DMA,
            pltpu.SemaphoreType.DMA,
        ],
    )
    def gather2(table, rowm, colm, out_r, out_c, idxr, idxc,
                br0, bc0, br1, bc1, sr0, sc0, sr1, sc1):
        wid = lax.axis_index("s") * NC + lax.axis_index("c")
        r0 = wid * R
        pltpu.sync_copy(rowm.at[pl.ds(r0, R)], idxr)
        pltpu.sync_copy(colm.at[pl.ds(r0, R)], idxc)

        def issue(g, br, bc, sr, sc):
            pltpu.async_copy(table.at[idxr.at[g]], br, sr)
            pltpu.async_copy(table.at[idxc.at[g]], bc, sc)

        def drain(br, bc, sr, sc):
            pltpu.make_async_copy(table.at[idxr.at[0]], br, sr).wait()
            pltpu.make_async_copy(table.at[idxc.at[0]], bc, sc).wait()

        issue(0, br0, bc0, sr0, sc0)

        def body(g2, carry):
            g = g2 * 2
            issue(g + 1, br1, bc1, sr1, sc1)
            drain(br0, bc0, sr0, sc0)
            e = (r0 + g) * C
            pltpu.sync_copy(br0, out_r.at[pl.ds(e, C)])
            pltpu.sync_copy(bc0, out_c.at[pl.ds(e, C)])

            @pl.when(g + 2 < R)
            def _():
                issue(g + 2, br0, bc0, sr0, sc0)

            drain(br1, bc1, sr1, sc1)
            e1 = (r0 + g + 1) * C
            pltpu.sync_copy(br1, out_r.at[pl.ds(e1, C)])
            pltpu.sync_copy(bc1, out_c.at[pl.ds(e1, C)])
            return carry

        lax.fori_loop(0, R // 2, body, 0)

    return gather2


_gather2_h = _make_gather2(H)
_gather2_pos = _make_gather2(H)


HW = H // 2
_RSC = E_PAD // NS // C
_IDXW = 40
_NODES_PER_TILE = N_PAD // NS


@functools.partial(
    pl.kernel,
    mesh=_MESH,
    compiler_params=_SC_PARAMS,
    out_type=(
        jax.ShapeDtypeStruct((N_PAD, HW), jnp.float32),
        jax.ShapeDtypeStruct((N_PAD, HW), jnp.float32),
    ),
    scratch_types=[
        pltpu.VMEM((_IDXW, C), jnp.int32),
        pltpu.VMEM((C, HW), jnp.float32),
        pltpu.VMEM((C, HW), jnp.float32),
        pltpu.VMEM_SHARED((N_PAD, HW), jnp.float32),
        pltpu.SemaphoreType.DMA,
        pltpu.SemaphoreType.DMA,
    ],
)
def _scatter_add(mv, rowm, zeros, agg0, agg1, idxv, mb0, mb1, shared,
                 sm0, sm1):
    cid = lax.axis_index("c")
    sid = lax.axis_index("s")
    z0 = sid * _NODES_PER_TILE
    pltpu.sync_copy(zeros.at[pl.ds(z0, _NODES_PER_TILE)],
                    shared.at[pl.ds(z0, _NODES_PER_TILE)])
    plsc.subcore_barrier()

    r0 = sid * _RSC

    def main(coff):
        def window(w, carry):
            rw = r0 + w * _IDXW
            pltpu.sync_copy(rowm.at[pl.ds(rw, _IDXW)], idxv)
            pltpu.async_copy(mv.at[pl.ds(rw * C, C), pl.ds(coff, HW)],
                             mb0, sm0)

            def chunk2(j2, carry2):
                j = j2 * 2
                pltpu.async_copy(
                    mv.at[pl.ds((rw + j + 1) * C, C), pl.ds(coff, HW)],
                    mb1, sm1)
                pltpu.make_async_copy(
                    mv.at[pl.ds(0, C), pl.ds(coff, HW)], mb0, sm0).wait()
                pltpu.sync_copy(mb0, shared.at[idxv.at[j]], add=True)

                @pl.when(j + 2 < _IDXW)
                def _():
                    pltpu.async_copy(
                        mv.at[pl.ds((rw + j + 2) * C, C), pl.ds(coff, HW)],
                        mb0, sm0)

                pltpu.make_async_copy(
                    mv.at[pl.ds(0, C), pl.ds(coff, HW)], mb1, sm1).wait()
                pltpu.sync_copy(mb1, shared.at[idxv.at[j + 1]], add=True)
                return carry2

            return lax.fori_loop(0, _IDXW // 2, chunk2, carry)

        lax.fori_loop(0, _RSC // _IDXW, window, 0)

    @pl.when(cid == 0)
    def _():
        main(0)

    @pl.when(cid == 1)
    def _():
        main(HW)

    plsc.subcore_barrier()

    @pl.when(cid == 0)
    def _():
        pltpu.sync_copy(shared.at[pl.ds(z0, _NODES_PER_TILE)],
                        agg0.at[pl.ds(z0, _NODES_PER_TILE)])

    @pl.when(cid == 1)
    def _():
        pltpu.sync_copy(shared.at[pl.ds(z0, _NODES_PER_TILE)],
                        agg1.at[pl.ds(z0, _NODES_PER_TILE)])


BE = 4096
BN = 1024


def _emb_body(z_ref, emb_ref, out_ref):
    z = z_ref[...]
    oh = (z == lax.broadcasted_iota(jnp.int32, (BN, 128), 1)).astype(jnp.float32)
    out_ref[...] = jnp.dot(oh, emb_ref[...], preferred_element_type=jnp.float32)


def _emb_lookup(z2d, emb_pad):
    return pl.pallas_call(
        _emb_body,
        grid=(N_PAD // BN,),
        in_specs=[
            pl.BlockSpec((BN, 1), lambda i: (i, 0)),
            pl.BlockSpec((128, H), lambda i: (0, 0)),
        ],
        out_specs=pl.BlockSpec((BN, H), lambda i: (i, 0)),
        out_shape=jax.ShapeDtypeStruct((N_PAD, H), jnp.float32),
    )(z2d, emb_pad)


def _dsq_body(prp_ref, pcp_ref, out_ref):
    rel = prp_ref[...] - pcp_ref[...]
    sq = rel * rel
    s0 = jnp.sum(sq[:, :H], axis=1, keepdims=True)
    s1 = jnp.sum(sq[:, H:], axis=1, keepdims=True)
    out_ref[...] = jnp.concatenate([s0, s1], axis=1)


def _dsq(prp, pcp):
    return pl.pallas_call(
        _dsq_body,
        grid=(E_PAD // BE,),
        in_specs=[
            pl.BlockSpec((BE // 2, 2 * H), lambda i: (i, 0)),
            pl.BlockSpec((BE // 2, 2 * H), lambda i: (i, 0)),
        ],
        out_specs=pl.BlockSpec((BE // 2, 2), lambda i: (i, 0)),
        out_shape=jax.ShapeDtypeStruct((E_PAD // 2, 2), jnp.float32),
    )(prp, pcp)


def _silu(x):
    return x * jax.nn.sigmoid(x)


def _bdot(a, b_ref):
    return jnp.dot(a.astype(jnp.bfloat16), b_ref[...],
                   preferred_element_type=jnp.float32)


BE2 = BE // 2


def _edge_body(hrp_ref, hcp_ref, dsqp_ref, w1rp_ref, w1cp_ref, w1d0_ref,
               w1d1_ref, b1p_ref, w2p_ref, b2p_ref, mp_ref):
    p = pl.program_id(0)
    dsqp = dsqp_ref[...]
    t = (_bdot(hrp_ref[...], w1rp_ref) + _bdot(hcp_ref[...], w1cp_ref)
         + dsqp[:, 0:1] * w1d0_ref[...]
         + dsqp[:, 1:2] * w1d1_ref[...]
         + b1p_ref[...])
    t = _silu(t)
    t = _bdot(t, w2p_ref) + b2p_ref[...]
    m = _silu(t)
    eid = (p * BE + 2 * lax.broadcasted_iota(jnp.int32, (BE2, 2 * H), 0)
           + (lax.broadcasted_iota(jnp.int32, (BE2, 2 * H), 1) >= H)
           .astype(jnp.int32))
    mp_ref[...] = jnp.where(eid < E, m, 0.0)


def _edge_mlp(hrp, hcp, dsqp, w1rp, w1cp, w1d0, w1d1, b1p, w2p, b2p):
    wspec = lambda shape: pl.BlockSpec(shape, lambda i: (0, 0))
    return pl.pallas_call(
        _edge_body,
        grid=(E_PAD // BE,),
        in_specs=[
            pl.BlockSpec((BE2, 2 * H), lambda i: (i, 0)),
            pl.BlockSpec((BE2, 2 * H), lambda i: (i, 0)),
            pl.BlockSpec((BE2, 2), lambda i: (i, 0)),
            wspec((2 * H, 2 * H)), wspec((2 * H, 2 * H)),
            wspec((1, 2 * H)), wspec((1, 2 * H)), wspec((1, 2 * H)),
            wspec((2 * H, 2 * H)), wspec((1, 2 * H)),
        ],
        out_specs=pl.BlockSpec((BE2, 2 * H), lambda i: (i, 0)),
        out_shape=jax.ShapeDtypeStruct((E_PAD // 2, 2 * H), jnp.float32),
    )(hrp, hcp, dsqp, w1rp, w1cp, w1d0, w1d1, b1p, w2p, b2p)


def _node_body(h_ref, a0_ref, a1_ref, wh_ref, wa0_ref, wa1_ref, b1_ref,
               w2_ref, b2_ref, g_ref, bb_ref, out_ref):
    h = h_ref[...]
    t = (jnp.dot(h, wh_ref[...], preferred_element_type=jnp.float32)
         + jnp.dot(a0_ref[...], wa0_ref[...], preferred_element_type=jnp.float32)
         + jnp.dot(a1_ref[...], wa1_ref[...], preferred_element_type=jnp.float32)
         + b1_ref[...])
    t = _silu(t)
    hn = jnp.dot(t, w2_ref[...], preferred_element_type=jnp.float32) + b2_ref[...]
    y = h + hn
    mu = jnp.mean(y, axis=-1, keepdims=True)
    d = y - mu
    var = jnp.mean(d * d, axis=-1, keepdims=True)
    out_ref[...] = d * lax.rsqrt(var + 1e-5) * g_ref[...] + bb_ref[...]


def _node_mlp(h, a0, a1, wh, wa0, wa1, b1, w2, b2, g, bb):
    wspec = lambda shape: pl.BlockSpec(shape, lambda i: (0, 0))
    return pl.pallas_call(
        _node_body,
        grid=(N_PAD // BN,),
        in_specs=[
            pl.BlockSpec((BN, H), lambda i: (i, 0)),
            pl.BlockSpec((BN, HW), lambda i: (i, 0)),
            pl.BlockSpec((BN, HW), lambda i: (i, 0)),
            wspec((H, H)), wspec((HW, H)), wspec((HW, H)), wspec((1, H)),
            wspec((H, H)), wspec((1, H)), wspec((1, H)), wspec((1, H)),
        ],
        out_specs=pl.BlockSpec((BN, H), lambda i: (i, 0)),
        out_shape=jax.ShapeDtypeStruct((N_PAD, H), jnp.float32),
    )(h, a0, a1, wh, wa0, wa1, b1, w2, b2, g, bb)


def kernel(z, pos, batch, edge_index, params):
    f32 = jnp.float32
    bf16 = jnp.bfloat16
    row = edge_index[0].astype(jnp.int32)
    col = edge_index[1].astype(jnp.int32)
    pad_e = E_PAD - E
    pad_idx = jnp.arange(pad_e, dtype=jnp.int32) % N
    row_p = jnp.concatenate([row, pad_idx]).reshape(E_PAD // C, C)
    col_p = jnp.concatenate([col, pad_idx]).reshape(E_PAD // C, C)

    z_p = jnp.concatenate(
        [z.astype(jnp.int32), jnp.zeros((N_PAD - N,), jnp.int32)]
    ).reshape(N_PAD, 1)
    pos64 = jnp.zeros((N_PAD, H), f32).at[:N, :3].set(pos.astype(f32))
    emb_pad = jnp.zeros((128, H), f32).at[:MAXZ].set(params["emb"].astype(f32))
    zeros_half = jnp.zeros((N_PAD, HW), f32)

    h = _emb_lookup(z_p, emb_pad)
    pr, pc = _gather2_pos(pos64, row_p, col_p)
    dsqp = _dsq(pr.reshape(E_PAD // 2, 2 * H), pc.reshape(E_PAD // 2, 2 * H))

    def blockdiag2(w):
        z = jnp.zeros((2 * H, 2 * H), f32)
        return z.at[:H, :H].set(w).at[H:, H:].set(w).astype(bf16)

    for i in range(L):
        p = params[f"l{i}"]
        w1rp = blockdiag2(p["eW1"][:H])
        w1cp = blockdiag2(p["eW1"][H:2 * H])
        w1d = p["eW1"][2 * H:].reshape(1, H)
        zpad = jnp.zeros((1, H), f32)
        w1d0 = jnp.concatenate([w1d, zpad], axis=1)
        w1d1 = jnp.concatenate([zpad, w1d], axis=1)
        b1p = jnp.tile(p["eb1"].reshape(1, H), (1, 2))
        b2p = jnp.tile(p["eb2"].reshape(1, H), (1, 2))

        hr, hc = _gather2_h(h, row_p, col_p)
        hrp = hr.reshape(E_PAD // 2, 2 * H)
        hcp = hc.reshape(E_PAD // 2, 2 * H)
        mp = _edge_mlp(hrp, hcp, dsqp, w1rp, w1cp, w1d0, w1d1, b1p,
                       blockdiag2(p["eW2"]), b2p)
        a0, a1 = _scatter_add(mp.reshape(E_PAD, H), row_p, zeros_half)

        wh = p["nW1"][:H]
        wa0 = p["nW1"][H:H + HW]
        wa1 = p["nW1"][H + HW:]
        h = _node_mlp(h, a0, a1, wh, wa0, wa1, p["nb1"].reshape(1, H),
                      p["nW2"], p["nb2"].reshape(1, H), p["g"].reshape(1, H),
                      p["b"].reshape(1, H))

    return h[:N]


# speedup vs baseline: 2.9512x; 1.0019x over previous
import functools

import jax
import jax.numpy as jnp
from jax import lax
from jax.experimental import pallas as pl
from jax.experimental.pallas import tpu as pltpu
from jax.experimental.pallas import tpu_sc as plsc

N = 50000
E = 800000
H = 64
MAXZ = 100
L = 4

NC = 2
NS = 16
NW = NC * NS
C = 128

E_PAD = 819200
N_PAD = 50176

_MESH = plsc.VectorSubcoreMesh(
    core_axis_name="c", subcore_axis_name="s", num_cores=NC, num_subcores=NS
)
_SC_PARAMS = pltpu.CompilerParams(use_tc_tiling_on_sc=False)


def _make_gather2(D):
    R = E_PAD // NW // C
    out_shape = (E_PAD, D)

    @functools.partial(
        pl.kernel,
        mesh=_MESH,
        compiler_params=_SC_PARAMS,
        out_type=(
            jax.ShapeDtypeStruct(out_shape, jnp.float32),
            jax.ShapeDtypeStruct(out_shape, jnp.float32),
        ),
        scratch_types=[
            pltpu.VMEM((R, C), jnp.int32),
            pltpu.VMEM((R, C), jnp.int32),
            pltpu.VMEM((C, D), jnp.float32),
            pltpu.VMEM((C, D), jnp.float32),
            pltpu.VMEM((C, D), jnp.float32),
            pltpu.VMEM((C, D), jnp.float32),
            pltpu.SemaphoreType.DMA,
            pltpu.SemaphoreType.DMA,
            pltpu.SemaphoreType.DMA,
            pltpu.SemaphoreType.DMA,
        ],
    )
    def gather2(table, rowm, colm, out_r, out_c, idxr, idxc,
                br0, bc0, br1, bc1, sr0, sc0, sr1, sc1):
        wid = lax.axis_index("s") * NC + lax.axis_index("c")
        r0 = wid * R
        pltpu.sync_copy(rowm.at[pl.ds(r0, R)], idxr)
        pltpu.sync_copy(colm.at[pl.ds(r0, R)], idxc)

        def issue(g, br, bc, sr, sc):
            pltpu.async_copy(table.at[idxr.at[g]], br, sr)
            pltpu.async_copy(table.at[idxc.at[g]], bc, sc)

        def drain(br, bc, sr, sc):
            pltpu.make_async_copy(table.at[idxr.at[0]], br, sr).wait()
            pltpu.make_async_copy(table.at[idxc.at[0]], bc, sc).wait()

        issue(0, br0, bc0, sr0, sc0)

        def body(g2, carry):
            g = g2 * 2
            issue(g + 1, br1, bc1, sr1, sc1)
            drain(br0, bc0, sr0, sc0)
            e = (r0 + g) * C
            pltpu.sync_copy(br0, out_r.at[pl.ds(e, C)])
            pltpu.sync_copy(bc0, out_c.at[pl.ds(e, C)])

            @pl.when(g + 2 < R)
            def _():
                issue(g + 2, br0, bc0, sr0, sc0)

            drain(br1, bc1, sr1, sc1)
            e1 = (r0 + g + 1) * C
            pltpu.sync_copy(br1, out_r.at[pl.ds(e1, C)])
            pltpu.sync_copy(bc1, out_c.at[pl.ds(e1, C)])
            return carry

        lax.fori_loop(0, R // 2, body, 0)

    return gather2


_gather2_h = _make_gather2(H)
_gather2_pos = _make_gather2(H)


HW = H // 2
_RSC = E_PAD // NS // C
_IDXW = 40
_NODES_PER_TILE = N_PAD // NS


@functools.partial(
    pl.kernel,
    mesh=_MESH,
    compiler_params=_SC_PARAMS,
    out_type=(
        jax.ShapeDtypeStruct((N_PAD, HW), jnp.float32),
        jax.ShapeDtypeStruct((N_PAD, HW), jnp.float32),
    ),
    scratch_types=[
        pltpu.VMEM((_IDXW, C), jnp.int32),
        pltpu.VMEM((C, HW), jnp.float32),
        pltpu.VMEM((C, HW), jnp.float32),
        pltpu.VMEM_SHARED((N_PAD, HW), jnp.float32),
        pltpu.SemaphoreType.DMA,
        pltpu.SemaphoreType.DMA,
    ],
)
def _scatter_add(mv, rowm, zeros, agg0, agg1, idxv, mb0, mb1, shared,
                 sm0, sm1):
    cid = lax.axis_index("c")
    sid = lax.axis_index("s")
    z0 = sid * _NODES_PER_TILE
    pltpu.sync_copy(zeros.at[pl.ds(z0, _NODES_PER_TILE)],
                    shared.at[pl.ds(z0, _NODES_PER_TILE)])
    plsc.subcore_barrier()

    r0 = sid * _RSC

    def main(coff):
        def window(w, carry):
            rw = r0 + w * _IDXW
            pltpu.sync_copy(rowm.at[pl.ds(rw, _IDXW)], idxv)
            pltpu.async_copy(mv.at[pl.ds(rw * C, C), pl.ds(coff, HW)],
                             mb0, sm0)

            def chunk2(j2, carry2):
                j = j2 * 2
                pltpu.async_copy(
                    mv.at[pl.ds((rw + j + 1) * C, C), pl.ds(coff, HW)],
                    mb1, sm1)
                pltpu.make_async_copy(
                    mv.at[pl.ds(0, C), pl.ds(coff, HW)], mb0, sm0).wait()
                pltpu.sync_copy(mb0, shared.at[idxv.at[j]], add=True)

                @pl.when(j + 2 < _IDXW)
                def _():
                    pltpu.async_copy(
                        mv.at[pl.ds((rw + j + 2) * C, C), pl.ds(coff, HW)],
                        mb0, sm0)

                pltpu.make_async_copy(
                    mv.at[pl.ds(0, C), pl.ds(coff, HW)], mb1, sm1).wait()
                pltpu.sync_copy(mb1, shared.at[idxv.at[j + 1]], add=True)
                return carry2

            return lax.fori_loop(0, _IDXW // 2, chunk2, carry)

        lax.fori_loop(0, _RSC // _IDXW, window, 0)

    @pl.when(cid == 0)
    def _():
        main(0)

    @pl.when(cid == 1)
    def _():
        main(HW)

    plsc.subcore_barrier()

    @pl.when(cid == 0)
    def _():
        pltpu.sync_copy(shared.at[pl.ds(z0, _NODES_PER_TILE)],
                        agg0.at[pl.ds(z0, _NODES_PER_TILE)])

    @pl.when(cid == 1)
    def _():
        pltpu.sync_copy(shared.at[pl.ds(z0, _NODES_PER_TILE)],
                        agg1.at[pl.ds(z0, _NODES_PER_TILE)])


BE = 4096
BN = 1024


def _emb_body(z_ref, emb_ref, out_ref):
    z = z_ref[...]
    oh = (z == lax.broadcasted_iota(jnp.int32, (BN, 128), 1)).astype(jnp.float32)
    out_ref[...] = jnp.dot(oh, emb_ref[...], preferred_element_type=jnp.float32)


def _emb_lookup(z2d, emb_pad):
    return pl.pallas_call(
        _emb_body,
        grid=(N_PAD // BN,),
        in_specs=[
            pl.BlockSpec((BN, 1), lambda i: (i, 0)),
            pl.BlockSpec((128, H), lambda i: (0, 0)),
        ],
        out_specs=pl.BlockSpec((BN, H), lambda i: (i, 0)),
        out_shape=jax.ShapeDtypeStruct((N_PAD, H), jnp.float32),
    )(z2d, emb_pad)


def _dsq_body(prp_ref, pcp_ref, out_ref):
    rel = prp_ref[...] - pcp_ref[...]
    sq = rel * rel
    s0 = jnp.sum(sq[:, :H], axis=1, keepdims=True)
    s1 = jnp.sum(sq[:, H:], axis=1, keepdims=True)
    out_ref[...] = jnp.concatenate([s0, s1], axis=1)


def _dsq(prp, pcp):
    return pl.pallas_call(
        _dsq_body,
        grid=(E_PAD // BE,),
        in_specs=[
            pl.BlockSpec((BE // 2, 2 * H), lambda i: (i, 0)),
            pl.BlockSpec((BE // 2, 2 * H), lambda i: (i, 0)),
        ],
        out_specs=pl.BlockSpec((BE // 2, 2), lambda i: (i, 0)),
        out_shape=jax.ShapeDtypeStruct((E_PAD // 2, 2), jnp.float32),
    )(prp, pcp)


def _silu(x):
    return x * jax.nn.sigmoid(x)


def _bdot(a, b_ref):
    return jnp.dot(a.astype(jnp.bfloat16), b_ref[...],
                   preferred_element_type=jnp.float32)


BE2 = BE // 2


def _edge_body(hrp_ref, hcp_ref, dsqp_ref, w1rp_ref, w1cp_ref, w1d0_ref,
               w1d1_ref, b1p_ref, w2p_ref, b2p_ref, mp_ref):
    p = pl.program_id(0)
    dsqp = dsqp_ref[...]
    t = (_bdot(hrp_ref[...], w1rp_ref) + _bdot(hcp_ref[...], w1cp_ref)
         + dsqp[:, 0:1] * w1d0_ref[...]
         + dsqp[:, 1:2] * w1d1_ref[...]
         + b1p_ref[...])
    t = _silu(t)
    t = _bdot(t, w2p_ref) + b2p_ref[...]
    m = _silu(t)
    eid = (p * BE + 2 * lax.broadcasted_iota(jnp.int32, (BE2, 2 * H), 0)
           + (lax.broadcasted_iota(jnp.int32, (BE2, 2 * H), 1) >= H)
           .astype(jnp.int32))
    mp_ref[...] = jnp.where(eid < E, m, 0.0)


def _edge_body0(hrp_ref, hcp_ref, prp_ref, pcp_ref, w1rp_ref, w1cp_ref,
                w1d0_ref, w1d1_ref, b1p_ref, w2p_ref, b2p_ref, mp_ref,
                dsqp_ref):
    p = pl.program_id(0)
    rel = prp_ref[...] - pcp_ref[...]
    sq = rel * rel
    s0 = jnp.sum(sq[:, :H], axis=1, keepdims=True)
    s1 = jnp.sum(sq[:, H:], axis=1, keepdims=True)
    dsqp_ref[...] = jnp.concatenate([s0, s1], axis=1)
    t = (_bdot(hrp_ref[...], w1rp_ref) + _bdot(hcp_ref[...], w1cp_ref)
         + s0 * w1d0_ref[...]
         + s1 * w1d1_ref[...]
         + b1p_ref[...])
    t = _silu(t)
    t = _bdot(t, w2p_ref) + b2p_ref[...]
    m = _silu(t)
    eid = (p * BE + 2 * lax.broadcasted_iota(jnp.int32, (BE2, 2 * H), 0)
           + (lax.broadcasted_iota(jnp.int32, (BE2, 2 * H), 1) >= H)
           .astype(jnp.int32))
    mp_ref[...] = jnp.where(eid < E, m, 0.0)


def _edge_mlp0(hrp, hcp, prp, pcp, w1rp, w1cp, w1d0, w1d1, b1p, w2p, b2p):
    wspec = lambda shape: pl.BlockSpec(shape, lambda i: (0, 0))
    return pl.pallas_call(
        _edge_body0,
        grid=(E_PAD // BE,),
        in_specs=[
            pl.BlockSpec((BE2, 2 * H), lambda i: (i, 0)),
            pl.BlockSpec((BE2, 2 * H), lambda i: (i, 0)),
            pl.BlockSpec((BE2, 2 * H), lambda i: (i, 0)),
            pl.BlockSpec((BE2, 2 * H), lambda i: (i, 0)),
            wspec((2 * H, 2 * H)), wspec((2 * H, 2 * H)),
            wspec((1, 2 * H)), wspec((1, 2 * H)), wspec((1, 2 * H)),
            wspec((2 * H, 2 * H)), wspec((1, 2 * H)),
        ],
        out_specs=(
            pl.BlockSpec((BE2, 2 * H), lambda i: (i, 0)),
            pl.BlockSpec((BE2, 2), lambda i: (i, 0)),
        ),
        out_shape=(
            jax.ShapeDtypeStruct((E_PAD // 2, 2 * H), jnp.float32),
            jax.ShapeDtypeStruct((E_PAD // 2, 2), jnp.float32),
        ),
    )(hrp, hcp, prp, pcp, w1rp, w1cp, w1d0, w1d1, b1p, w2p, b2p)


def _edge_mlp(hrp, hcp, dsqp, w1rp, w1cp, w1d0, w1d1, b1p, w2p, b2p):
    wspec = lambda shape: pl.BlockSpec(shape, lambda i: (0, 0))
    return pl.pallas_call(
        _edge_body,
        grid=(E_PAD // BE,),
        in_specs=[
            pl.BlockSpec((BE2, 2 * H), lambda i: (i, 0)),
            pl.BlockSpec((BE2, 2 * H), lambda i: (i, 0)),
            pl.BlockSpec((BE2, 2), lambda i: (i, 0)),
            wspec((2 * H, 2 * H)), wspec((2 * H, 2 * H)),
            wspec((1, 2 * H)), wspec((1, 2 * H)), wspec((1, 2 * H)),
            wspec((2 * H, 2 * H)), wspec((1, 2 * H)),
        ],
        out_specs=pl.BlockSpec((BE2, 2 * H), lambda i: (i, 0)),
        out_shape=jax.ShapeDtypeStruct((E_PAD // 2, 2 * H), jnp.float32),
    )(hrp, hcp, dsqp, w1rp, w1cp, w1d0, w1d1, b1p, w2p, b2p)


def _node_body(h_ref, a0_ref, a1_ref, wh_ref, wa0_ref, wa1_ref, b1_ref,
               w2_ref, b2_ref, g_ref, bb_ref, out_ref):
    h = h_ref[...]
    t = (jnp.dot(h, wh_ref[...], preferred_element_type=jnp.float32)
         + jnp.dot(a0_ref[...], wa0_ref[...], preferred_element_type=jnp.float32)
         + jnp.dot(a1_ref[...], wa1_ref[...], preferred_element_type=jnp.float32)
         + b1_ref[...])
    t = _silu(t)
    hn = jnp.dot(t, w2_ref[...], preferred_element_type=jnp.float32) + b2_ref[...]
    y = h + hn
    mu = jnp.mean(y, axis=-1, keepdims=True)
    d = y - mu
    var = jnp.mean(d * d, axis=-1, keepdims=True)
    out_ref[...] = d * lax.rsqrt(var + 1e-5) * g_ref[...] + bb_ref[...]


def _node_mlp(h, a0, a1, wh, wa0, wa1, b1, w2, b2, g, bb):
    wspec = lambda shape: pl.BlockSpec(shape, lambda i: (0, 0))
    return pl.pallas_call(
        _node_body,
        grid=(N_PAD // BN,),
        in_specs=[
            pl.BlockSpec((BN, H), lambda i: (i, 0)),
            pl.BlockSpec((BN, HW), lambda i: (i, 0)),
            pl.BlockSpec((BN, HW), lambda i: (i, 0)),
            wspec((H, H)), wspec((HW, H)), wspec((HW, H)), wspec((1, H)),
            wspec((H, H)), wspec((1, H)), wspec((1, H)), wspec((1, H)),
        ],
        out_specs=pl.BlockSpec((BN, H), lambda i: (i, 0)),
        out_shape=jax.ShapeDtypeStruct((N_PAD, H), jnp.float32),
    )(h, a0, a1, wh, wa0, wa1, b1, w2, b2, g, bb)


def kernel(z, pos, batch, edge_index, params):
    f32 = jnp.float32
    bf16 = jnp.bfloat16
    row = edge_index[0].astype(jnp.int32)
    col = edge_index[1].astype(jnp.int32)
    pad_e = E_PAD - E
    pad_idx = jnp.arange(pad_e, dtype=jnp.int32) % N
    row_p = jnp.concatenate([row, pad_idx]).reshape(E_PAD // C, C)
    col_p = jnp.concatenate([col, pad_idx]).reshape(E_PAD // C, C)

    z_p = jnp.concatenate(
        [z.astype(jnp.int32), jnp.zeros((N_PAD - N,), jnp.int32)]
    ).reshape(N_PAD, 1)
    pos64 = jnp.zeros((N_PAD, H), f32).at[:N, :3].set(pos.astype(f32))
    emb_pad = jnp.zeros((128, H), f32).at[:MAXZ].set(params["emb"].astype(f32))
    zeros_half = jnp.zeros((N_PAD, HW), f32)

    h = _emb_lookup(z_p, emb_pad)
    pr, pc = _gather2_pos(pos64, row_p, col_p)
    prp = pr.reshape(E_PAD // 2, 2 * H)
    pcp = pc.reshape(E_PAD // 2, 2 * H)
    dsqp = None

    def blockdiag2(w):
        z = jnp.zeros((2 * H, 2 * H), f32)
        return z.at[:H, :H].set(w).at[H:, H:].set(w).astype(bf16)

    for i in range(L):
        p = params[f"l{i}"]
        w1rp = blockdiag2(p["eW1"][:H])
        w1cp = blockdiag2(p["eW1"][H:2 * H])
        w1d = p["eW1"][2 * H:].reshape(1, H)
        zpad = jnp.zeros((1, H), f32)
        w1d0 = jnp.concatenate([w1d, zpad], axis=1)
        w1d1 = jnp.concatenate([zpad, w1d], axis=1)
        b1p = jnp.tile(p["eb1"].reshape(1, H), (1, 2))
        b2p = jnp.tile(p["eb2"].reshape(1, H), (1, 2))

        hr, hc = _gather2_h(h, row_p, col_p)
        hrp = hr.reshape(E_PAD // 2, 2 * H)
        hcp = hc.reshape(E_PAD // 2, 2 * H)
        if i == 0:
            mp, dsqp = _edge_mlp0(hrp, hcp, prp, pcp, w1rp, w1cp, w1d0,
                                  w1d1, b1p, blockdiag2(p["eW2"]), b2p)
        else:
            mp = _edge_mlp(hrp, hcp, dsqp, w1rp, w1cp, w1d0, w1d1, b1p,
                           blockdiag2(p["eW2"]), b2p)
        a0, a1 = _scatter_add(mp.reshape(E_PAD, H), row_p, zeros_half)

        wh = p["nW1"][:H]
        wa0 = p["nW1"][H:H + HW]
        wa1 = p["nW1"][H + HW:]
        h = _node_mlp(h, a0, a1, wh, wa0, wa1, p["nb1"].reshape(1, H),
                      p["nW2"], p["nb2"].reshape(1, H), p["g"].reshape(1, H),
                      p["b"].reshape(1, H))

    return h[:N]


# speedup vs baseline: 3.1340x; 1.0619x over previous
import functools

import jax
import jax.numpy as jnp
from jax import lax
from jax.experimental import pallas as pl
from jax.experimental.pallas import tpu as pltpu
from jax.experimental.pallas import tpu_sc as plsc

N = 50000
E = 800000
H = 64
MAXZ = 100
L = 4

NC = 2
NS = 16
NW = NC * NS
C = 128

E_PAD = 819200
N_PAD = 50176

_MESH = plsc.VectorSubcoreMesh(
    core_axis_name="c", subcore_axis_name="s", num_cores=NC, num_subcores=NS
)
_SC_PARAMS = pltpu.CompilerParams(use_tc_tiling_on_sc=False)


def _make_gather2(D):
    R = E_PAD // NW // C
    out_shape = (E_PAD, D)

    @functools.partial(
        pl.kernel,
        mesh=_MESH,
        compiler_params=_SC_PARAMS,
        out_type=(
            jax.ShapeDtypeStruct(out_shape, jnp.float32),
            jax.ShapeDtypeStruct(out_shape, jnp.float32),
        ),
        scratch_types=[
            pltpu.VMEM((R, C), jnp.int32),
            pltpu.VMEM((R, C), jnp.int32),
            pltpu.VMEM((C, D), jnp.float32),
            pltpu.VMEM((C, D), jnp.float32),
            pltpu.VMEM((C, D), jnp.float32),
            pltpu.VMEM((C, D), jnp.float32),
            pltpu.SemaphoreType.DMA,
            pltpu.SemaphoreType.DMA,
            pltpu.SemaphoreType.DMA,
            pltpu.SemaphoreType.DMA,
        ],
    )
    def gather2(table, rowm, colm, out_r, out_c, idxr, idxc,
                br0, bc0, br1, bc1, sr0, sc0, sr1, sc1):
        wid = lax.axis_index("s") * NC + lax.axis_index("c")
        r0 = wid * R
        pltpu.sync_copy(rowm.at[pl.ds(r0, R)], idxr)
        pltpu.sync_copy(colm.at[pl.ds(r0, R)], idxc)

        def issue(g, br, bc, sr, sc):
            pltpu.async_copy(table.at[idxr.at[g]], br, sr)
            pltpu.async_copy(table.at[idxc.at[g]], bc, sc)

        def drain(br, bc, sr, sc):
            pltpu.make_async_copy(table.at[idxr.at[0]], br, sr).wait()
            pltpu.make_async_copy(table.at[idxc.at[0]], bc, sc).wait()

        issue(0, br0, bc0, sr0, sc0)

        def body(g2, carry):
            g = g2 * 2
            issue(g + 1, br1, bc1, sr1, sc1)
            drain(br0, bc0, sr0, sc0)
            e = (r0 + g) * C
            pltpu.sync_copy(br0, out_r.at[pl.ds(e, C)])
            pltpu.sync_copy(bc0, out_c.at[pl.ds(e, C)])

            @pl.when(g + 2 < R)
            def _():
                issue(g + 2, br0, bc0, sr0, sc0)

            drain(br1, bc1, sr1, sc1)
            e1 = (r0 + g + 1) * C
            pltpu.sync_copy(br1, out_r.at[pl.ds(e1, C)])
            pltpu.sync_copy(bc1, out_c.at[pl.ds(e1, C)])
            return carry

        lax.fori_loop(0, R // 2, body, 0)

    return gather2


_gather2_h = _make_gather2(H)
_gather2_pos = _make_gather2(H)


HW = H // 2
_RSC = E_PAD // NS // C
_IDXW = 40
_NODES_PER_TILE = N_PAD // NS


@functools.partial(
    pl.kernel,
    mesh=_MESH,
    compiler_params=_SC_PARAMS,
    out_type=(
        jax.ShapeDtypeStruct((N_PAD, HW), jnp.float32),
        jax.ShapeDtypeStruct((N_PAD, HW), jnp.float32),
    ),
    scratch_types=[
        pltpu.VMEM((_IDXW, C), jnp.int32),
        pltpu.VMEM((C, HW), jnp.float32),
        pltpu.VMEM((C, HW), jnp.float32),
        pltpu.VMEM_SHARED((N_PAD, HW), jnp.float32),
        pltpu.SemaphoreType.DMA,
        pltpu.SemaphoreType.DMA,
    ],
)
def _scatter_add(mv, rowm, zeros, agg0, agg1, idxv, mb0, mb1, shared,
                 sm0, sm1):
    cid = lax.axis_index("c")
    sid = lax.axis_index("s")
    z0 = sid * _NODES_PER_TILE
    pltpu.sync_copy(zeros.at[pl.ds(z0, _NODES_PER_TILE)],
                    shared.at[pl.ds(z0, _NODES_PER_TILE)])
    plsc.subcore_barrier()

    r0 = sid * _RSC

    def main(coff):
        def window(w, carry):
            rw = r0 + w * _IDXW
            pltpu.sync_copy(rowm.at[pl.ds(rw, _IDXW)], idxv)
            pltpu.async_copy(mv.at[pl.ds(rw * C, C), pl.ds(coff, HW)],
                             mb0, sm0)

            def chunk2(j2, carry2):
                j = j2 * 2
                pltpu.async_copy(
                    mv.at[pl.ds((rw + j + 1) * C, C), pl.ds(coff, HW)],
                    mb1, sm1)
                pltpu.make_async_copy(
                    mv.at[pl.ds(0, C), pl.ds(coff, HW)], mb0, sm0).wait()
                pltpu.sync_copy(mb0, shared.at[idxv.at[j]], add=True)

                @pl.when(j + 2 < _IDXW)
                def _():
                    pltpu.async_copy(
                        mv.at[pl.ds((rw + j + 2) * C, C), pl.ds(coff, HW)],
                        mb0, sm0)

                pltpu.make_async_copy(
                    mv.at[pl.ds(0, C), pl.ds(coff, HW)], mb1, sm1).wait()
                pltpu.sync_copy(mb1, shared.at[idxv.at[j + 1]], add=True)
                return carry2

            return lax.fori_loop(0, _IDXW // 2, chunk2, carry)

        lax.fori_loop(0, _RSC // _IDXW, window, 0)

    @pl.when(cid == 0)
    def _():
        main(0)

    @pl.when(cid == 1)
    def _():
        main(HW)

    plsc.subcore_barrier()

    @pl.when(cid == 0)
    def _():
        pltpu.sync_copy(shared.at[pl.ds(z0, _NODES_PER_TILE)],
                        agg0.at[pl.ds(z0, _NODES_PER_TILE)])

    @pl.when(cid == 1)
    def _():
        pltpu.sync_copy(shared.at[pl.ds(z0, _NODES_PER_TILE)],
                        agg1.at[pl.ds(z0, _NODES_PER_TILE)])


BE = 8192
BN = 1024


def _emb_body(z_ref, emb_ref, out_ref):
    z = z_ref[...]
    oh = (z == lax.broadcasted_iota(jnp.int32, (BN, 128), 1)).astype(jnp.float32)
    out_ref[...] = jnp.dot(oh, emb_ref[...], preferred_element_type=jnp.float32)


def _emb_lookup(z2d, emb_pad):
    return pl.pallas_call(
        _emb_body,
        grid=(N_PAD // BN,),
        in_specs=[
            pl.BlockSpec((BN, 1), lambda i: (i, 0)),
            pl.BlockSpec((128, H), lambda i: (0, 0)),
        ],
        out_specs=pl.BlockSpec((BN, H), lambda i: (i, 0)),
        out_shape=jax.ShapeDtypeStruct((N_PAD, H), jnp.float32),
    )(z2d, emb_pad)


def _dsq_body(prp_ref, pcp_ref, out_ref):
    rel = prp_ref[...] - pcp_ref[...]
    sq = rel * rel
    s0 = jnp.sum(sq[:, :H], axis=1, keepdims=True)
    s1 = jnp.sum(sq[:, H:], axis=1, keepdims=True)
    out_ref[...] = jnp.concatenate([s0, s1], axis=1)


def _dsq(prp, pcp):
    return pl.pallas_call(
        _dsq_body,
        grid=(E_PAD // BE,),
        in_specs=[
            pl.BlockSpec((BE // 2, 2 * H), lambda i: (i, 0)),
            pl.BlockSpec((BE // 2, 2 * H), lambda i: (i, 0)),
        ],
        out_specs=pl.BlockSpec((BE // 2, 2), lambda i: (i, 0)),
        out_shape=jax.ShapeDtypeStruct((E_PAD // 2, 2), jnp.float32),
    )(prp, pcp)


def _silu(x):
    return x * jax.nn.sigmoid(x)


def _bdot(a, b_ref):
    return jnp.dot(a.astype(jnp.bfloat16), b_ref[...],
                   preferred_element_type=jnp.float32)


BE2 = BE // 2


def _edge_body(hrp_ref, hcp_ref, dsqp_ref, w1rp_ref, w1cp_ref, w1d0_ref,
               w1d1_ref, b1p_ref, w2p_ref, b2p_ref, mp_ref):
    p = pl.program_id(0)
    dsqp = dsqp_ref[...]
    t = (_bdot(hrp_ref[...], w1rp_ref) + _bdot(hcp_ref[...], w1cp_ref)
         + dsqp[:, 0:1] * w1d0_ref[...]
         + dsqp[:, 1:2] * w1d1_ref[...]
         + b1p_ref[...])
    t = _silu(t)
    t = _bdot(t, w2p_ref) + b2p_ref[...]
    m = _silu(t)
    eid = (p * BE + 2 * lax.broadcasted_iota(jnp.int32, (BE2, 2 * H), 0)
           + (lax.broadcasted_iota(jnp.int32, (BE2, 2 * H), 1) >= H)
           .astype(jnp.int32))
    mp_ref[...] = jnp.where(eid < E, m, 0.0)


def _edge_body0(hrp_ref, hcp_ref, prp_ref, pcp_ref, w1rp_ref, w1cp_ref,
                w1d0_ref, w1d1_ref, b1p_ref, w2p_ref, b2p_ref, mp_ref,
                dsqp_ref):
    p = pl.program_id(0)
    rel = prp_ref[...] - pcp_ref[...]
    sq = rel * rel
    s0 = jnp.sum(sq[:, :H], axis=1, keepdims=True)
    s1 = jnp.sum(sq[:, H:], axis=1, keepdims=True)
    dsqp_ref[...] = jnp.concatenate([s0, s1], axis=1)
    t = (_bdot(hrp_ref[...], w1rp_ref) + _bdot(hcp_ref[...], w1cp_ref)
         + s0 * w1d0_ref[...]
         + s1 * w1d1_ref[...]
         + b1p_ref[...])
    t = _silu(t)
    t = _bdot(t, w2p_ref) + b2p_ref[...]
    m = _silu(t)
    eid = (p * BE + 2 * lax.broadcasted_iota(jnp.int32, (BE2, 2 * H), 0)
           + (lax.broadcasted_iota(jnp.int32, (BE2, 2 * H), 1) >= H)
           .astype(jnp.int32))
    mp_ref[...] = jnp.where(eid < E, m, 0.0)


def _edge_mlp0(hrp, hcp, prp, pcp, w1rp, w1cp, w1d0, w1d1, b1p, w2p, b2p):
    wspec = lambda shape: pl.BlockSpec(shape, lambda i: (0, 0))
    return pl.pallas_call(
        _edge_body0,
        grid=(E_PAD // BE,),
        in_specs=[
            pl.BlockSpec((BE2, 2 * H), lambda i: (i, 0)),
            pl.BlockSpec((BE2, 2 * H), lambda i: (i, 0)),
            pl.BlockSpec((BE2, 2 * H), lambda i: (i, 0)),
            pl.BlockSpec((BE2, 2 * H), lambda i: (i, 0)),
            wspec((2 * H, 2 * H)), wspec((2 * H, 2 * H)),
            wspec((1, 2 * H)), wspec((1, 2 * H)), wspec((1, 2 * H)),
            wspec((2 * H, 2 * H)), wspec((1, 2 * H)),
        ],
        out_specs=(
            pl.BlockSpec((BE2, 2 * H), lambda i: (i, 0)),
            pl.BlockSpec((BE2, 2), lambda i: (i, 0)),
        ),
        out_shape=(
            jax.ShapeDtypeStruct((E_PAD // 2, 2 * H), jnp.float32),
            jax.ShapeDtypeStruct((E_PAD // 2, 2), jnp.float32),
        ),
    )(hrp, hcp, prp, pcp, w1rp, w1cp, w1d0, w1d1, b1p, w2p, b2p)


def _edge_mlp(hrp, hcp, dsqp, w1rp, w1cp, w1d0, w1d1, b1p, w2p, b2p):
    wspec = lambda shape: pl.BlockSpec(shape, lambda i: (0, 0))
    return pl.pallas_call(
        _edge_body,
        grid=(E_PAD // BE,),
        in_specs=[
            pl.BlockSpec((BE2, 2 * H), lambda i: (i, 0)),
            pl.BlockSpec((BE2, 2 * H), lambda i: (i, 0)),
            pl.BlockSpec((BE2, 2), lambda i: (i, 0)),
            wspec((2 * H, 2 * H)), wspec((2 * H, 2 * H)),
            wspec((1, 2 * H)), wspec((1, 2 * H)), wspec((1, 2 * H)),
            wspec((2 * H, 2 * H)), wspec((1, 2 * H)),
        ],
        out_specs=pl.BlockSpec((BE2, 2 * H), lambda i: (i, 0)),
        out_shape=jax.ShapeDtypeStruct((E_PAD // 2, 2 * H), jnp.float32),
    )(hrp, hcp, dsqp, w1rp, w1cp, w1d0, w1d1, b1p, w2p, b2p)


def _node_body(h_ref, a0_ref, a1_ref, wh_ref, wa0_ref, wa1_ref, b1_ref,
               w2_ref, b2_ref, g_ref, bb_ref, out_ref):
    h = h_ref[...]
    t = (jnp.dot(h, wh_ref[...], preferred_element_type=jnp.float32)
         + jnp.dot(a0_ref[...], wa0_ref[...], preferred_element_type=jnp.float32)
         + jnp.dot(a1_ref[...], wa1_ref[...], preferred_element_type=jnp.float32)
         + b1_ref[...])
    t = _silu(t)
    hn = jnp.dot(t, w2_ref[...], preferred_element_type=jnp.float32) + b2_ref[...]
    y = h + hn
    mu = jnp.mean(y, axis=-1, keepdims=True)
    d = y - mu
    var = jnp.mean(d * d, axis=-1, keepdims=True)
    out_ref[...] = d * lax.rsqrt(var + 1e-5) * g_ref[...] + bb_ref[...]


def _node_mlp(h, a0, a1, wh, wa0, wa1, b1, w2, b2, g, bb):
    wspec = lambda shape: pl.BlockSpec(shape, lambda i: (0, 0))
    return pl.pallas_call(
        _node_body,
        grid=(N_PAD // BN,),
        in_specs=[
            pl.BlockSpec((BN, H), lambda i: (i, 0)),
            pl.BlockSpec((BN, HW), lambda i: (i, 0)),
            pl.BlockSpec((BN, HW), lambda i: (i, 0)),
            wspec((H, H)), wspec((HW, H)), wspec((HW, H)), wspec((1, H)),
            wspec((H, H)), wspec((1, H)), wspec((1, H)), wspec((1, H)),
        ],
        out_specs=pl.BlockSpec((BN, H), lambda i: (i, 0)),
        out_shape=jax.ShapeDtypeStruct((N_PAD, H), jnp.float32),
    )(h, a0, a1, wh, wa0, wa1, b1, w2, b2, g, bb)


def kernel(z, pos, batch, edge_index, params):
    f32 = jnp.float32
    bf16 = jnp.bfloat16
    row = edge_index[0].astype(jnp.int32)
    col = edge_index[1].astype(jnp.int32)
    pad_e = E_PAD - E
    pad_idx = jnp.arange(pad_e, dtype=jnp.int32) % N
    row_p = jnp.concatenate([row, pad_idx]).reshape(E_PAD // C, C)
    col_p = jnp.concatenate([col, pad_idx]).reshape(E_PAD // C, C)

    z_p = jnp.concatenate(
        [z.astype(jnp.int32), jnp.zeros((N_PAD - N,), jnp.int32)]
    ).reshape(N_PAD, 1)
    pos64 = jnp.zeros((N_PAD, H), f32).at[:N, :3].set(pos.astype(f32))
    emb_pad = jnp.zeros((128, H), f32).at[:MAXZ].set(params["emb"].astype(f32))
    zeros_half = jnp.zeros((N_PAD, HW), f32)

    h = _emb_lookup(z_p, emb_pad)
    pr, pc = _gather2_pos(pos64, row_p, col_p)
    prp = pr.reshape(E_PAD // 2, 2 * H)
    pcp = pc.reshape(E_PAD // 2, 2 * H)
    dsqp = None

    def blockdiag2(w):
        z = jnp.zeros((2 * H, 2 * H), f32)
        return z.at[:H, :H].set(w).at[H:, H:].set(w).astype(bf16)

    for i in range(L):
        p = params[f"l{i}"]
        w1rp = blockdiag2(p["eW1"][:H])
        w1cp = blockdiag2(p["eW1"][H:2 * H])
        w1d = p["eW1"][2 * H:].reshape(1, H)
        zpad = jnp.zeros((1, H), f32)
        w1d0 = jnp.concatenate([w1d, zpad], axis=1)
        w1d1 = jnp.concatenate([zpad, w1d], axis=1)
        b1p = jnp.tile(p["eb1"].reshape(1, H), (1, 2))
        b2p = jnp.tile(p["eb2"].reshape(1, H), (1, 2))

        hr, hc = _gather2_h(h, row_p, col_p)
        hrp = hr.reshape(E_PAD // 2, 2 * H)
        hcp = hc.reshape(E_PAD // 2, 2 * H)
        if i == 0:
            mp, dsqp = _edge_mlp0(hrp, hcp, prp, pcp, w1rp, w1cp, w1d0,
                                  w1d1, b1p, blockdiag2(p["eW2"]), b2p)
        else:
            mp = _edge_mlp(hrp, hcp, dsqp, w1rp, w1cp, w1d0, w1d1, b1p,
                           blockdiag2(p["eW2"]), b2p)
        a0, a1 = _scatter_add(mp.reshape(E_PAD, H), row_p, zeros_half)

        wh = p["nW1"][:H]
        wa0 = p["nW1"][H:H + HW]
        wa1 = p["nW1"][H + HW:]
        h = _node_mlp(h, a0, a1, wh, wa0, wa1, p["nb1"].reshape(1, H),
                      p["nW2"], p["nb2"].reshape(1, H), p["g"].reshape(1, H),
                      p["b"].reshape(1, H))

    return h[:N]


# speedup vs baseline: 3.2079x; 1.0236x over previous
import functools

import jax
import jax.numpy as jnp
from jax import lax
from jax.experimental import pallas as pl
from jax.experimental.pallas import tpu as pltpu
from jax.experimental.pallas import tpu_sc as plsc

N = 50000
E = 800000
H = 64
MAXZ = 100
L = 4

NC = 2
NS = 16
NW = NC * NS
C = 128

E_PAD = 819200
N_PAD = 50176

_MESH = plsc.VectorSubcoreMesh(
    core_axis_name="c", subcore_axis_name="s", num_cores=NC, num_subcores=NS
)
_SC_PARAMS = pltpu.CompilerParams(use_tc_tiling_on_sc=False)


def _make_gather2(D):
    R = E_PAD // NW // C
    out_shape = (E_PAD, D)

    @functools.partial(
        pl.kernel,
        mesh=_MESH,
        compiler_params=_SC_PARAMS,
        out_type=(
            jax.ShapeDtypeStruct(out_shape, jnp.float32),
            jax.ShapeDtypeStruct(out_shape, jnp.float32),
        ),
        scratch_types=[
            pltpu.VMEM((R, C), jnp.int32),
            pltpu.VMEM((R, C), jnp.int32),
            pltpu.VMEM((C, D), jnp.float32),
            pltpu.VMEM((C, D), jnp.float32),
            pltpu.VMEM((C, D), jnp.float32),
            pltpu.VMEM((C, D), jnp.float32),
            pltpu.SemaphoreType.DMA,
            pltpu.SemaphoreType.DMA,
            pltpu.SemaphoreType.DMA,
            pltpu.SemaphoreType.DMA,
        ],
    )
    def gather2(table, rowm, colm, out_r, out_c, idxr, idxc,
                br0, bc0, br1, bc1, sr0, sc0, sr1, sc1):
        wid = lax.axis_index("s") * NC + lax.axis_index("c")
        r0 = wid * R
        pltpu.sync_copy(rowm.at[pl.ds(r0, R)], idxr)
        pltpu.sync_copy(colm.at[pl.ds(r0, R)], idxc)

        def issue(g, br, bc, sr, sc):
            pltpu.async_copy(table.at[idxr.at[g]], br, sr)
            pltpu.async_copy(table.at[idxc.at[g]], bc, sc)

        def drain(br, bc, sr, sc):
            pltpu.make_async_copy(table.at[idxr.at[0]], br, sr).wait()
            pltpu.make_async_copy(table.at[idxc.at[0]], bc, sc).wait()

        issue(0, br0, bc0, sr0, sc0)

        def body(g2, carry):
            g = g2 * 2
            issue(g + 1, br1, bc1, sr1, sc1)
            drain(br0, bc0, sr0, sc0)
            e = (r0 + g) * C
            pltpu.sync_copy(br0, out_r.at[pl.ds(e, C)])
            pltpu.sync_copy(bc0, out_c.at[pl.ds(e, C)])

            @pl.when(g + 2 < R)
            def _():
                issue(g + 2, br0, bc0, sr0, sc0)

            drain(br1, bc1, sr1, sc1)
            e1 = (r0 + g + 1) * C
            pltpu.sync_copy(br1, out_r.at[pl.ds(e1, C)])
            pltpu.sync_copy(bc1, out_c.at[pl.ds(e1, C)])
            return carry

        lax.fori_loop(0, R // 2, body, 0)

    return gather2


_gather2_h = _make_gather2(H)
_gather2_pos = _make_gather2(H)


HW = H // 2
_RSC = E_PAD // NS // C
_IDXW = 40
_NODES_PER_TILE = N_PAD // NS


@functools.partial(
    pl.kernel,
    mesh=_MESH,
    compiler_params=_SC_PARAMS,
    out_type=(
        jax.ShapeDtypeStruct((N_PAD, HW), jnp.float32),
        jax.ShapeDtypeStruct((N_PAD, HW), jnp.float32),
    ),
    scratch_types=[
        pltpu.VMEM((_IDXW, C), jnp.int32),
        pltpu.VMEM((C, HW), jnp.float32),
        pltpu.VMEM((C, HW), jnp.float32),
        pltpu.VMEM_SHARED((N_PAD, HW), jnp.float32),
        pltpu.SemaphoreType.DMA,
        pltpu.SemaphoreType.DMA,
    ],
)
def _scatter_add(mv, rowm, zeros, agg0, agg1, idxv, mb0, mb1, shared,
                 sm0, sm1):
    cid = lax.axis_index("c")
    sid = lax.axis_index("s")
    z0 = sid * _NODES_PER_TILE
    pltpu.sync_copy(zeros.at[pl.ds(z0, _NODES_PER_TILE)],
                    shared.at[pl.ds(z0, _NODES_PER_TILE)])
    plsc.subcore_barrier()

    r0 = sid * _RSC

    def main(coff):
        def window(w, carry):
            rw = r0 + w * _IDXW
            pltpu.sync_copy(rowm.at[pl.ds(rw, _IDXW)], idxv)
            pltpu.async_copy(mv.at[pl.ds(rw * C, C), pl.ds(coff, HW)],
                             mb0, sm0)

            def chunk2(j2, carry2):
                j = j2 * 2
                pltpu.async_copy(
                    mv.at[pl.ds((rw + j + 1) * C, C), pl.ds(coff, HW)],
                    mb1, sm1)
                pltpu.make_async_copy(
                    mv.at[pl.ds(0, C), pl.ds(coff, HW)], mb0, sm0).wait()
                pltpu.sync_copy(mb0, shared.at[idxv.at[j]], add=True)

                @pl.when(j + 2 < _IDXW)
                def _():
                    pltpu.async_copy(
                        mv.at[pl.ds((rw + j + 2) * C, C), pl.ds(coff, HW)],
                        mb0, sm0)

                pltpu.make_async_copy(
                    mv.at[pl.ds(0, C), pl.ds(coff, HW)], mb1, sm1).wait()
                pltpu.sync_copy(mb1, shared.at[idxv.at[j + 1]], add=True)
                return carry2

            return lax.fori_loop(0, _IDXW // 2, chunk2, carry)

        lax.fori_loop(0, _RSC // _IDXW, window, 0)

    @pl.when(cid == 0)
    def _():
        main(0)

    @pl.when(cid == 1)
    def _():
        main(HW)

    plsc.subcore_barrier()

    @pl.when(cid == 0)
    def _():
        pltpu.sync_copy(shared.at[pl.ds(z0, _NODES_PER_TILE)],
                        agg0.at[pl.ds(z0, _NODES_PER_TILE)])

    @pl.when(cid == 1)
    def _():
        pltpu.sync_copy(shared.at[pl.ds(z0, _NODES_PER_TILE)],
                        agg1.at[pl.ds(z0, _NODES_PER_TILE)])


BE = 16384
BE0 = 8192
BN = 1024


def _emb_body(z_ref, emb_ref, out_ref):
    z = z_ref[...]
    oh = (z == lax.broadcasted_iota(jnp.int32, (BN, 128), 1)).astype(jnp.float32)
    out_ref[...] = jnp.dot(oh, emb_ref[...], preferred_element_type=jnp.float32)


def _emb_lookup(z2d, emb_pad):
    return pl.pallas_call(
        _emb_body,
        grid=(N_PAD // BN,),
        in_specs=[
            pl.BlockSpec((BN, 1), lambda i: (i, 0)),
            pl.BlockSpec((128, H), lambda i: (0, 0)),
        ],
        out_specs=pl.BlockSpec((BN, H), lambda i: (i, 0)),
        out_shape=jax.ShapeDtypeStruct((N_PAD, H), jnp.float32),
    )(z2d, emb_pad)


def _dsq_body(prp_ref, pcp_ref, out_ref):
    rel = prp_ref[...] - pcp_ref[...]
    sq = rel * rel
    s0 = jnp.sum(sq[:, :H], axis=1, keepdims=True)
    s1 = jnp.sum(sq[:, H:], axis=1, keepdims=True)
    out_ref[...] = jnp.concatenate([s0, s1], axis=1)


def _dsq(prp, pcp):
    return pl.pallas_call(
        _dsq_body,
        grid=(E_PAD // BE,),
        in_specs=[
            pl.BlockSpec((BE // 2, 2 * H), lambda i: (i, 0)),
            pl.BlockSpec((BE // 2, 2 * H), lambda i: (i, 0)),
        ],
        out_specs=pl.BlockSpec((BE // 2, 2), lambda i: (i, 0)),
        out_shape=jax.ShapeDtypeStruct((E_PAD // 2, 2), jnp.float32),
    )(prp, pcp)


def _silu(x):
    return x * jax.nn.sigmoid(x)


def _bdot(a, b_ref):
    return jnp.dot(a.astype(jnp.bfloat16), b_ref[...],
                   preferred_element_type=jnp.float32)


BE2 = BE // 2


def _edge_body(hrp_ref, hcp_ref, dsqp_ref, w1rp_ref, w1cp_ref, w1d0_ref,
               w1d1_ref, b1p_ref, w2p_ref, b2p_ref, mp_ref):
    p = pl.program_id(0)
    dsqp = dsqp_ref[...]
    t = (_bdot(hrp_ref[...], w1rp_ref) + _bdot(hcp_ref[...], w1cp_ref)
         + dsqp[:, 0:1] * w1d0_ref[...]
         + dsqp[:, 1:2] * w1d1_ref[...]
         + b1p_ref[...])
    t = _silu(t)
    t = _bdot(t, w2p_ref) + b2p_ref[...]
    m = _silu(t)
    eid = (p * BE + 2 * lax.broadcasted_iota(jnp.int32, (BE2, 2 * H), 0)
           + (lax.broadcasted_iota(jnp.int32, (BE2, 2 * H), 1) >= H)
           .astype(jnp.int32))
    mp_ref[...] = jnp.where(eid < E, m, 0.0)


def _edge_body0(hrp_ref, hcp_ref, prp_ref, pcp_ref, w1rp_ref, w1cp_ref,
                w1d0_ref, w1d1_ref, b1p_ref, w2p_ref, b2p_ref, mp_ref,
                dsqp_ref):
    p = pl.program_id(0)
    BE, BE2 = BE0, BE0 // 2
    rel = prp_ref[...] - pcp_ref[...]
    sq = rel * rel
    s0 = jnp.sum(sq[:, :H], axis=1, keepdims=True)
    s1 = jnp.sum(sq[:, H:], axis=1, keepdims=True)
    dsqp_ref[...] = jnp.concatenate([s0, s1], axis=1)
    t = (_bdot(hrp_ref[...], w1rp_ref) + _bdot(hcp_ref[...], w1cp_ref)
         + s0 * w1d0_ref[...]
         + s1 * w1d1_ref[...]
         + b1p_ref[...])
    t = _silu(t)
    t = _bdot(t, w2p_ref) + b2p_ref[...]
    m = _silu(t)
    eid = (p * BE + 2 * lax.broadcasted_iota(jnp.int32, (BE2, 2 * H), 0)
           + (lax.broadcasted_iota(jnp.int32, (BE2, 2 * H), 1) >= H)
           .astype(jnp.int32))
    mp_ref[...] = jnp.where(eid < E, m, 0.0)


def _edge_mlp0(hrp, hcp, prp, pcp, w1rp, w1cp, w1d0, w1d1, b1p, w2p, b2p):
    wspec = lambda shape: pl.BlockSpec(shape, lambda i: (0, 0))
    BE2 = BE0 // 2
    return pl.pallas_call(
        _edge_body0,
        grid=(E_PAD // BE0,),
        in_specs=[
            pl.BlockSpec((BE2, 2 * H), lambda i: (i, 0)),
            pl.BlockSpec((BE2, 2 * H), lambda i: (i, 0)),
            pl.BlockSpec((BE2, 2 * H), lambda i: (i, 0)),
            pl.BlockSpec((BE2, 2 * H), lambda i: (i, 0)),
            wspec((2 * H, 2 * H)), wspec((2 * H, 2 * H)),
            wspec((1, 2 * H)), wspec((1, 2 * H)), wspec((1, 2 * H)),
            wspec((2 * H, 2 * H)), wspec((1, 2 * H)),
        ],
        out_specs=(
            pl.BlockSpec((BE2, 2 * H), lambda i: (i, 0)),
            pl.BlockSpec((BE2, 2), lambda i: (i, 0)),
        ),
        out_shape=(
            jax.ShapeDtypeStruct((E_PAD // 2, 2 * H), jnp.float32),
            jax.ShapeDtypeStruct((E_PAD // 2, 2), jnp.float32),
        ),
    )(hrp, hcp, prp, pcp, w1rp, w1cp, w1d0, w1d1, b1p, w2p, b2p)


def _edge_mlp(hrp, hcp, dsqp, w1rp, w1cp, w1d0, w1d1, b1p, w2p, b2p):
    wspec = lambda shape: pl.BlockSpec(shape, lambda i: (0, 0))
    return pl.pallas_call(
        _edge_body,
        grid=(E_PAD // BE,),
        in_specs=[
            pl.BlockSpec((BE2, 2 * H), lambda i: (i, 0)),
            pl.BlockSpec((BE2, 2 * H), lambda i: (i, 0)),
            pl.BlockSpec((BE2, 2), lambda i: (i, 0)),
            wspec((2 * H, 2 * H)), wspec((2 * H, 2 * H)),
            wspec((1, 2 * H)), wspec((1, 2 * H)), wspec((1, 2 * H)),
            wspec((2 * H, 2 * H)), wspec((1, 2 * H)),
        ],
        out_specs=pl.BlockSpec((BE2, 2 * H), lambda i: (i, 0)),
        out_shape=jax.ShapeDtypeStruct((E_PAD // 2, 2 * H), jnp.float32),
    )(hrp, hcp, dsqp, w1rp, w1cp, w1d0, w1d1, b1p, w2p, b2p)


def _node_body(h_ref, a0_ref, a1_ref, wh_ref, wa0_ref, wa1_ref, b1_ref,
               w2_ref, b2_ref, g_ref, bb_ref, out_ref):
    h = h_ref[...]
    t = (jnp.dot(h, wh_ref[...], preferred_element_type=jnp.float32)
         + jnp.dot(a0_ref[...], wa0_ref[...], preferred_element_type=jnp.float32)
         + jnp.dot(a1_ref[...], wa1_ref[...], preferred_element_type=jnp.float32)
         + b1_ref[...])
    t = _silu(t)
    hn = jnp.dot(t, w2_ref[...], preferred_element_type=jnp.float32) + b2_ref[...]
    y = h + hn
    mu = jnp.mean(y, axis=-1, keepdims=True)
    d = y - mu
    var = jnp.mean(d * d, axis=-1, keepdims=True)
    out_ref[...] = d * lax.rsqrt(var + 1e-5) * g_ref[...] + bb_ref[...]


def _node_mlp(h, a0, a1, wh, wa0, wa1, b1, w2, b2, g, bb):
    wspec = lambda shape: pl.BlockSpec(shape, lambda i: (0, 0))
    return pl.pallas_call(
        _node_body,
        grid=(N_PAD // BN,),
        in_specs=[
            pl.BlockSpec((BN, H), lambda i: (i, 0)),
            pl.BlockSpec((BN, HW), lambda i: (i, 0)),
            pl.BlockSpec((BN, HW), lambda i: (i, 0)),
            wspec((H, H)), wspec((HW, H)), wspec((HW, H)), wspec((1, H)),
            wspec((H, H)), wspec((1, H)), wspec((1, H)), wspec((1, H)),
        ],
        out_specs=pl.BlockSpec((BN, H), lambda i: (i, 0)),
        out_shape=jax.ShapeDtypeStruct((N_PAD, H), jnp.float32),
    )(h, a0, a1, wh, wa0, wa1, b1, w2, b2, g, bb)


def kernel(z, pos, batch, edge_index, params):
    f32 = jnp.float32
    bf16 = jnp.bfloat16
    row = edge_index[0].astype(jnp.int32)
    col = edge_index[1].astype(jnp.int32)
    pad_e = E_PAD - E
    pad_idx = jnp.arange(pad_e, dtype=jnp.int32) % N
    row_p = jnp.concatenate([row, pad_idx]).reshape(E_PAD // C, C)
    col_p = jnp.concatenate([col, pad_idx]).reshape(E_PAD // C, C)

    z_p = jnp.concatenate(
        [z.astype(jnp.int32), jnp.zeros((N_PAD - N,), jnp.int32)]
    ).reshape(N_PAD, 1)
    pos64 = jnp.zeros((N_PAD, H), f32).at[:N, :3].set(pos.astype(f32))
    emb_pad = jnp.zeros((128, H), f32).at[:MAXZ].set(params["emb"].astype(f32))
    zeros_half = jnp.zeros((N_PAD, HW), f32)

    h = _emb_lookup(z_p, emb_pad)
    pr, pc = _gather2_pos(pos64, row_p, col_p)
    prp = pr.reshape(E_PAD // 2, 2 * H)
    pcp = pc.reshape(E_PAD // 2, 2 * H)
    dsqp = None

    def blockdiag2(w):
        z = jnp.zeros((2 * H, 2 * H), f32)
        return z.at[:H, :H].set(w).at[H:, H:].set(w).astype(bf16)

    for i in range(L):
        p = params[f"l{i}"]
        w1rp = blockdiag2(p["eW1"][:H])
        w1cp = blockdiag2(p["eW1"][H:2 * H])
        w1d = p["eW1"][2 * H:].reshape(1, H)
        zpad = jnp.zeros((1, H), f32)
        w1d0 = jnp.concatenate([w1d, zpad], axis=1)
        w1d1 = jnp.concatenate([zpad, w1d], axis=1)
        b1p = jnp.tile(p["eb1"].reshape(1, H), (1, 2))
        b2p = jnp.tile(p["eb2"].reshape(1, H), (1, 2))

        hr, hc = _gather2_h(h, row_p, col_p)
        hrp = hr.reshape(E_PAD // 2, 2 * H)
        hcp = hc.reshape(E_PAD // 2, 2 * H)
        if i == 0:
            mp, dsqp = _edge_mlp0(hrp, hcp, prp, pcp, w1rp, w1cp, w1d0,
                                  w1d1, b1p, blockdiag2(p["eW2"]), b2p)
        else:
            mp = _edge_mlp(hrp, hcp, dsqp, w1rp, w1cp, w1d0, w1d1, b1p,
                           blockdiag2(p["eW2"]), b2p)
        a0, a1 = _scatter_add(mp.reshape(E_PAD, H), row_p, zeros_half)

        wh = p["nW1"][:H]
        wa0 = p["nW1"][H:H + HW]
        wa1 = p["nW1"][H + HW:]
        h = _node_mlp(h, a0, a1, wh, wa0, wa1, p["nb1"].reshape(1, H),
                      p["nW2"], p["nb2"].reshape(1, H), p["g"].reshape(1, H),
                      p["b"].reshape(1, H))

    return h[:N]


# speedup vs baseline: 3.2083x; 1.0001x over previous
import functools

import jax
import jax.numpy as jnp
from jax import lax
from jax.experimental import pallas as pl
from jax.experimental.pallas import tpu as pltpu
from jax.experimental.pallas import tpu_sc as plsc

N = 50000
E = 800000
H = 64
MAXZ = 100
L = 4

NC = 2
NS = 16
NW = NC * NS
C = 128

E_PAD = 819200
N_PAD = 50176

_MESH = plsc.VectorSubcoreMesh(
    core_axis_name="c", subcore_axis_name="s", num_cores=NC, num_subcores=NS
)
_SC_PARAMS = pltpu.CompilerParams(use_tc_tiling_on_sc=False)


def _make_gather2(D):
    R = E_PAD // NW // C
    out_shape = (E_PAD, D)

    @functools.partial(
        pl.kernel,
        mesh=_MESH,
        compiler_params=_SC_PARAMS,
        out_type=(
            jax.ShapeDtypeStruct(out_shape, jnp.float32),
            jax.ShapeDtypeStruct(out_shape, jnp.float32),
        ),
        scratch_types=[
            pltpu.VMEM((R, C), jnp.int32),
            pltpu.VMEM((R, C), jnp.int32),
            pltpu.VMEM((C, D), jnp.float32),
            pltpu.VMEM((C, D), jnp.float32),
            pltpu.VMEM((C, D), jnp.float32),
            pltpu.VMEM((C, D), jnp.float32),
            pltpu.SemaphoreType.DMA,
            pltpu.SemaphoreType.DMA,
            pltpu.SemaphoreType.DMA,
            pltpu.SemaphoreType.DMA,
        ],
    )
    def gather2(table, rowm, colm, out_r, out_c, idxr, idxc,
                br0, bc0, br1, bc1, sr0, sc0, sr1, sc1):
        wid = lax.axis_index("s") * NC + lax.axis_index("c")
        r0 = wid * R
        pltpu.sync_copy(rowm.at[pl.ds(r0, R)], idxr)
        pltpu.sync_copy(colm.at[pl.ds(r0, R)], idxc)

        def issue(g, br, bc, sr, sc):
            pltpu.async_copy(table.at[idxr.at[g]], br, sr)
            pltpu.async_copy(table.at[idxc.at[g]], bc, sc)

        def drain(br, bc, sr, sc):
            pltpu.make_async_copy(table.at[idxr.at[0]], br, sr).wait()
            pltpu.make_async_copy(table.at[idxc.at[0]], bc, sc).wait()

        issue(0, br0, bc0, sr0, sc0)

        def body(g2, carry):
            g = g2 * 2
            issue(g + 1, br1, bc1, sr1, sc1)
            drain(br0, bc0, sr0, sc0)
            e = (r0 + g) * C
            pltpu.sync_copy(br0, out_r.at[pl.ds(e, C)])
            pltpu.sync_copy(bc0, out_c.at[pl.ds(e, C)])

            @pl.when(g + 2 < R)
            def _():
                issue(g + 2, br0, bc0, sr0, sc0)

            drain(br1, bc1, sr1, sc1)
            e1 = (r0 + g + 1) * C
            pltpu.sync_copy(br1, out_r.at[pl.ds(e1, C)])
            pltpu.sync_copy(bc1, out_c.at[pl.ds(e1, C)])
            return carry

        lax.fori_loop(0, R // 2, body, 0)

    return gather2


_gather2_h = _make_gather2(H)
_gather2_pos = _make_gather2(H)


HW = H // 2
_RSC = E_PAD // NS // C
_IDXW = 40
_NODES_PER_TILE = N_PAD // NS


@functools.partial(
    pl.kernel,
    mesh=_MESH,
    compiler_params=_SC_PARAMS,
    out_type=(
        jax.ShapeDtypeStruct((N_PAD, HW), jnp.float32),
        jax.ShapeDtypeStruct((N_PAD, HW), jnp.float32),
    ),
    scratch_types=[
        pltpu.VMEM((_IDXW, C), jnp.int32),
        pltpu.VMEM((C, HW), jnp.float32),
        pltpu.VMEM((C, HW), jnp.float32),
        pltpu.VMEM_SHARED((N_PAD, HW), jnp.float32),
        pltpu.SemaphoreType.DMA,
        pltpu.SemaphoreType.DMA,
    ],
)
def _scatter_add(mv, rowm, zeros, agg0, agg1, idxv, mb0, mb1, shared,
                 sm0, sm1):
    cid = lax.axis_index("c")
    sid = lax.axis_index("s")
    z0 = sid * _NODES_PER_TILE
    pltpu.sync_copy(zeros.at[pl.ds(z0, _NODES_PER_TILE)],
                    shared.at[pl.ds(z0, _NODES_PER_TILE)])
    plsc.subcore_barrier()

    r0 = sid * _RSC

    def main(coff):
        def window(w, carry):
            rw = r0 + w * _IDXW
            pltpu.sync_copy(rowm.at[pl.ds(rw, _IDXW)], idxv)
            pltpu.async_copy(mv.at[pl.ds(rw * C, C), pl.ds(coff, HW)],
                             mb0, sm0)

            def chunk2(j2, carry2):
                j = j2 * 2
                pltpu.async_copy(
                    mv.at[pl.ds((rw + j + 1) * C, C), pl.ds(coff, HW)],
                    mb1, sm1)
                pltpu.make_async_copy(
                    mv.at[pl.ds(0, C), pl.ds(coff, HW)], mb0, sm0).wait()
                pltpu.sync_copy(mb0, shared.at[idxv.at[j]], add=True)

                @pl.when(j + 2 < _IDXW)
                def _():
                    pltpu.async_copy(
                        mv.at[pl.ds((rw + j + 2) * C, C), pl.ds(coff, HW)],
                        mb0, sm0)

                pltpu.make_async_copy(
                    mv.at[pl.ds(0, C), pl.ds(coff, HW)], mb1, sm1).wait()
                pltpu.sync_copy(mb1, shared.at[idxv.at[j + 1]], add=True)
                return carry2

            return lax.fori_loop(0, _IDXW // 2, chunk2, carry)

        lax.fori_loop(0, _RSC // _IDXW, window, 0)

    @pl.when(cid == 0)
    def _():
        main(0)

    @pl.when(cid == 1)
    def _():
        main(HW)

    plsc.subcore_barrier()

    @pl.when(cid == 0)
    def _():
        pltpu.sync_copy(shared.at[pl.ds(z0, _NODES_PER_TILE)],
                        agg0.at[pl.ds(z0, _NODES_PER_TILE)])

    @pl.when(cid == 1)
    def _():
        pltpu.sync_copy(shared.at[pl.ds(z0, _NODES_PER_TILE)],
                        agg1.at[pl.ds(z0, _NODES_PER_TILE)])


BE = 16384
BE0 = 8192
BN = 1024


def _emb_body(z_ref, emb_ref, out_ref):
    z = z_ref[...]
    oh = (z == lax.broadcasted_iota(jnp.int32, (BN, 128), 1)).astype(jnp.float32)
    out_ref[...] = jnp.dot(oh, emb_ref[...], preferred_element_type=jnp.float32)


def _emb_lookup(z2d, emb_pad):
    return pl.pallas_call(
        _emb_body,
        grid=(N_PAD // BN,),
        in_specs=[
            pl.BlockSpec((BN, 1), lambda i: (i, 0)),
            pl.BlockSpec((128, H), lambda i: (0, 0)),
        ],
        out_specs=pl.BlockSpec((BN, H), lambda i: (i, 0)),
        out_shape=jax.ShapeDtypeStruct((N_PAD, H), jnp.float32),
    )(z2d, emb_pad)


def _silu(x):
    return x * jax.nn.sigmoid(x)


def _bdot(a, b_ref):
    return jnp.dot(a.astype(jnp.bfloat16), b_ref[...],
                   preferred_element_type=jnp.float32)


BE2 = BE // 2


def _edge_body(hrp_ref, hcp_ref, dsqp_ref, w1rp_ref, w1cp_ref, w1d0_ref,
               w1d1_ref, b1p_ref, w2p_ref, b2p_ref, mp_ref):
    p = pl.program_id(0)
    dsqp = dsqp_ref[...]
    t = (_bdot(hrp_ref[...], w1rp_ref) + _bdot(hcp_ref[...], w1cp_ref)
         + dsqp[:, 0:1] * w1d0_ref[...]
         + dsqp[:, 1:2] * w1d1_ref[...]
         + b1p_ref[...])
    t = _silu(t)
    t = _bdot(t, w2p_ref) + b2p_ref[...]
    m = _silu(t)
    eid = (p * BE + 2 * lax.broadcasted_iota(jnp.int32, (BE2, 2 * H), 0)
           + (lax.broadcasted_iota(jnp.int32, (BE2, 2 * H), 1) >= H)
           .astype(jnp.int32))
    mp_ref[...] = jnp.where(eid < E, m, 0.0)


def _edge_body0(hrp_ref, hcp_ref, prp_ref, pcp_ref, w1rp_ref, w1cp_ref,
                w1d0_ref, w1d1_ref, b1p_ref, w2p_ref, b2p_ref, mp_ref,
                dsqp_ref):
    p = pl.program_id(0)
    BE, BE2 = BE0, BE0 // 2
    rel = prp_ref[...] - pcp_ref[...]
    sq = rel * rel
    s0 = jnp.sum(sq[:, :H], axis=1, keepdims=True)
    s1 = jnp.sum(sq[:, H:], axis=1, keepdims=True)
    dsqp_ref[...] = jnp.concatenate([s0, s1], axis=1)
    t = (_bdot(hrp_ref[...], w1rp_ref) + _bdot(hcp_ref[...], w1cp_ref)
         + s0 * w1d0_ref[...]
         + s1 * w1d1_ref[...]
         + b1p_ref[...])
    t = _silu(t)
    t = _bdot(t, w2p_ref) + b2p_ref[...]
    m = _silu(t)
    eid = (p * BE + 2 * lax.broadcasted_iota(jnp.int32, (BE2, 2 * H), 0)
           + (lax.broadcasted_iota(jnp.int32, (BE2, 2 * H), 1) >= H)
           .astype(jnp.int32))
    mp_ref[...] = jnp.where(eid < E, m, 0.0)


def _edge_mlp0(hrp, hcp, prp, pcp, w1rp, w1cp, w1d0, w1d1, b1p, w2p, b2p):
    wspec = lambda shape: pl.BlockSpec(shape, lambda i: (0, 0))
    BE2 = BE0 // 2
    return pl.pallas_call(
        _edge_body0,
        grid=(E_PAD // BE0,),
        in_specs=[
            pl.BlockSpec((BE2, 2 * H), lambda i: (i, 0)),
            pl.BlockSpec((BE2, 2 * H), lambda i: (i, 0)),
            pl.BlockSpec((BE2, 2 * H), lambda i: (i, 0)),
            pl.BlockSpec((BE2, 2 * H), lambda i: (i, 0)),
            wspec((2 * H, 2 * H)), wspec((2 * H, 2 * H)),
            wspec((1, 2 * H)), wspec((1, 2 * H)), wspec((1, 2 * H)),
            wspec((2 * H, 2 * H)), wspec((1, 2 * H)),
        ],
        out_specs=(
            pl.BlockSpec((BE2, 2 * H), lambda i: (i, 0)),
            pl.BlockSpec((BE2, 2), lambda i: (i, 0)),
        ),
        out_shape=(
            jax.ShapeDtypeStruct((E_PAD // 2, 2 * H), jnp.float32),
            jax.ShapeDtypeStruct((E_PAD // 2, 2), jnp.float32),
        ),
    )(hrp, hcp, prp, pcp, w1rp, w1cp, w1d0, w1d1, b1p, w2p, b2p)


def _edge_mlp(hrp, hcp, dsqp, w1rp, w1cp, w1d0, w1d1, b1p, w2p, b2p):
    wspec = lambda shape: pl.BlockSpec(shape, lambda i: (0, 0))
    return pl.pallas_call(
        _edge_body,
        grid=(E_PAD // BE,),
        in_specs=[
            pl.BlockSpec((BE2, 2 * H), lambda i: (i, 0)),
            pl.BlockSpec((BE2, 2 * H), lambda i: (i, 0)),
            pl.BlockSpec((BE2, 2), lambda i: (i, 0)),
            wspec((2 * H, 2 * H)), wspec((2 * H, 2 * H)),
            wspec((1, 2 * H)), wspec((1, 2 * H)), wspec((1, 2 * H)),
            wspec((2 * H, 2 * H)), wspec((1, 2 * H)),
        ],
        out_specs=pl.BlockSpec((BE2, 2 * H), lambda i: (i, 0)),
        out_shape=jax.ShapeDtypeStruct((E_PAD // 2, 2 * H), jnp.float32),
    )(hrp, hcp, dsqp, w1rp, w1cp, w1d0, w1d1, b1p, w2p, b2p)


def _node_body(h_ref, a0_ref, a1_ref, wh_ref, wa0_ref, wa1_ref, b1_ref,
               w2_ref, b2_ref, g_ref, bb_ref, out_ref):
    h = h_ref[...]
    t = (jnp.dot(h, wh_ref[...], preferred_element_type=jnp.float32)
         + jnp.dot(a0_ref[...], wa0_ref[...], preferred_element_type=jnp.float32)
         + jnp.dot(a1_ref[...], wa1_ref[...], preferred_element_type=jnp.float32)
         + b1_ref[...])
    t = _silu(t)
    hn = jnp.dot(t, w2_ref[...], preferred_element_type=jnp.float32) + b2_ref[...]
    y = h + hn
    mu = jnp.mean(y, axis=-1, keepdims=True)
    d = y - mu
    var = jnp.mean(d * d, axis=-1, keepdims=True)
    out_ref[...] = d * lax.rsqrt(var + 1e-5) * g_ref[...] + bb_ref[...]


def _node_mlp(h, a0, a1, wh, wa0, wa1, b1, w2, b2, g, bb):
    wspec = lambda shape: pl.BlockSpec(shape, lambda i: (0, 0))
    return pl.pallas_call(
        _node_body,
        grid=(N_PAD // BN,),
        in_specs=[
            pl.BlockSpec((BN, H), lambda i: (i, 0)),
            pl.BlockSpec((BN, HW), lambda i: (i, 0)),
            pl.BlockSpec((BN, HW), lambda i: (i, 0)),
            wspec((H, H)), wspec((HW, H)), wspec((HW, H)), wspec((1, H)),
            wspec((H, H)), wspec((1, H)), wspec((1, H)), wspec((1, H)),
        ],
        out_specs=pl.BlockSpec((BN, H), lambda i: (i, 0)),
        out_shape=jax.ShapeDtypeStruct((N_PAD, H), jnp.float32),
    )(h, a0, a1, wh, wa0, wa1, b1, w2, b2, g, bb)


def kernel(z, pos, batch, edge_index, params):
    f32 = jnp.float32
    bf16 = jnp.bfloat16
    row = edge_index[0].astype(jnp.int32)
    col = edge_index[1].astype(jnp.int32)
    pad_e = E_PAD - E
    pad_idx = jnp.arange(pad_e, dtype=jnp.int32) % N
    row_p = jnp.concatenate([row, pad_idx]).reshape(E_PAD // C, C)
    col_p = jnp.concatenate([col, pad_idx]).reshape(E_PAD // C, C)

    z_p = jnp.concatenate(
        [z.astype(jnp.int32), jnp.zeros((N_PAD - N,), jnp.int32)]
    ).reshape(N_PAD, 1)
    pos64 = jnp.zeros((N_PAD, H), f32).at[:N, :3].set(pos.astype(f32))
    emb_pad = jnp.zeros((128, H), f32).at[:MAXZ].set(params["emb"].astype(f32))
    zeros_half = jnp.zeros((N_PAD, HW), f32)

    h = _emb_lookup(z_p, emb_pad)
    pr, pc = _gather2_pos(pos64, row_p, col_p)
    prp = pr.reshape(E_PAD // 2, 2 * H)
    pcp = pc.reshape(E_PAD // 2, 2 * H)
    dsqp = None

    def blockdiag2(w):
        z = jnp.zeros((2 * H, 2 * H), f32)
        return z.at[:H, :H].set(w).at[H:, H:].set(w).astype(bf16)

    for i in range(L):
        p = params[f"l{i}"]
        w1rp = blockdiag2(p["eW1"][:H])
        w1cp = blockdiag2(p["eW1"][H:2 * H])
        w1d = p["eW1"][2 * H:].reshape(1, H)
        zpad = jnp.zeros((1, H), f32)
        w1d0 = jnp.concatenate([w1d, zpad], axis=1)
        w1d1 = jnp.concatenate([zpad, w1d], axis=1)
        b1p = jnp.tile(p["eb1"].reshape(1, H), (1, 2))
        b2p = jnp.tile(p["eb2"].reshape(1, H), (1, 2))

        hr, hc = _gather2_h(h, row_p, col_p)
        hrp = hr.reshape(E_PAD // 2, 2 * H)
        hcp = hc.reshape(E_PAD // 2, 2 * H)
        if i == 0:
            mp, dsqp = _edge_mlp0(hrp, hcp, prp, pcp, w1rp, w1cp, w1d0,
                                  w1d1, b1p, blockdiag2(p["eW2"]), b2p)
        else:
            mp = _edge_mlp(hrp, hcp, dsqp, w1rp, w1cp, w1d0, w1d1, b1p,
                           blockdiag2(p["eW2"]), b2p)
        a0, a1 = _scatter_add(mp.reshape(E_PAD, H), row_p, zeros_half)

        wh = p["nW1"][:H]
        wa0 = p["nW1"][H:H + HW]
        wa1 = p["nW1"][H + HW:]
        h = _node_mlp(h, a0, a1, wh, wa0, wa1, p["nb1"].reshape(1, H),
                      p["nW2"], p["nb2"].reshape(1, H), p["g"].reshape(1, H),
                      p["b"].reshape(1, H))

    return h[:N]
